# Initial kernel scaffold; baseline (speedup 1.0000x reference)
import functools

import jax
import jax.numpy as jnp
from jax import lax
from jax.experimental import pallas as pl
from jax.experimental.pallas import tpu as pltpu
from jax.experimental.pallas import tpu_sc as plsc

N = 10000
E = 320000
IN_DIM = 128
HIDDEN = 256
OUT_DIM = 128
B = 64

NSUB = 16
EDGES_PER_SUB = E // NSUB
DEG_CHUNK = 5000
DEG_NCH = EDGES_PER_SUB // DEG_CHUNK
K = 250
NCH = EDGES_PER_SUB // K
ROWS_PER_IO_SUB = 1000

_f32 = jnp.float32


def _vector_mesh():
    return plsc.VectorSubcoreMesh(core_axis_name="c", subcore_axis_name="s")


def _deg_body(dst_hbm, deg_hbm, ones_v, idx_v, acc_sh, sem):
    c = lax.axis_index("c")
    s = lax.axis_index("s")
    ones_v[...] = jnp.ones((DEG_CHUNK,), _f32)

    @pl.when((c == 0) & (s < 10))
    def _():
        pltpu.sync_copy(ones_v.at[pl.ds(0, ROWS_PER_IO_SUB)],
                        acc_sh.at[pl.ds(s * ROWS_PER_IO_SUB, ROWS_PER_IO_SUB)])

    plsc.subcore_barrier()

    @pl.when(c == 0)
    def _():
        @pl.loop(0, DEG_NCH)
        def _(j):
            pltpu.sync_copy(dst_hbm.at[s].at[j], idx_v)
            pltpu.sync_copy(ones_v, acc_sh.at[idx_v], add=True)

    plsc.subcore_barrier()

    @pl.when((c == 0) & (s < 10))
    def _():
        pltpu.sync_copy(acc_sh.at[pl.ds(s * ROWS_PER_IO_SUB, ROWS_PER_IO_SUB)],
                        deg_hbm.at[pl.ds(s * ROWS_PER_IO_SUB, ROWS_PER_IO_SUB)])


def _degrees(dst_d):
    kern = pl.kernel(
        _deg_body,
        out_type=jax.ShapeDtypeStruct((N,), _f32),
        mesh=_vector_mesh(),
        scratch_types=[
            pltpu.VMEM((DEG_CHUNK,), _f32),
            pltpu.VMEM((DEG_CHUNK,), jnp.int32),
            pltpu.VMEM_SHARED((N,), _f32),
            pltpu.SemaphoreType.DMA,
        ],
    )
    return kern(dst_d)


def _scatter_body(half, hs_hbm, src_hbm, dst_hbm, agg_hbm,
                  src_v, dst_v, data_v, acc_sh, sem):
    c = lax.axis_index("c")
    s = lax.axis_index("s")
    table = hs_hbm.at[c]

    @pl.when(s < 10)
    def _():
        rows = pl.ds(s * ROWS_PER_IO_SUB, ROWS_PER_IO_SUB)
        pltpu.sync_copy(table.at[rows], acc_sh.at[rows])

    pltpu.sync_copy(src_hbm.at[s], src_v)
    pltpu.sync_copy(dst_hbm.at[s], dst_v)
    plsc.subcore_barrier()

    @pl.loop(0, NCH)
    def _(j):
        pltpu.async_copy(table.at[src_v.at[j]], data_v, sem).wait()
        pltpu.sync_copy(data_v, acc_sh.at[dst_v.at[j]], add=True)

    plsc.subcore_barrier()

    @pl.when(s < 10)
    def _():
        rows = pl.ds(s * ROWS_PER_IO_SUB, ROWS_PER_IO_SUB)
        pltpu.sync_copy(acc_sh.at[rows], agg_hbm.at[c].at[rows])


def _edge_aggregate(hs, src_g, dst_g, half):
    kern = pl.kernel(
        functools.partial(_scatter_body, half),
        out_type=jax.ShapeDtypeStruct((2, N, half), _f32),
        mesh=_vector_mesh(),
        scratch_types=[
            pltpu.VMEM((NCH, K), jnp.int32),
            pltpu.VMEM((NCH, K), jnp.int32),
            pltpu.VMEM((K, half), _f32),
            pltpu.VMEM_SHARED((N, half), _f32),
            pltpu.SemaphoreType.DMA,
        ],
    )
    return kern(hs, src_g, dst_g)


def _mm1_body(x_ref, deg_ref, w_ref, out_ref):
    sc = lax.rsqrt(deg_ref[...])
    out_ref[0] = jnp.dot(x_ref[...] * sc, w_ref[...],
                         preferred_element_type=_f32,
                         precision=lax.Precision.HIGHEST)


def _mm1(x, deg2d, W1):
    grid = (N // 1000, 2)
    return pl.pallas_call(
        _mm1_body,
        grid=grid,
        in_specs=[
            pl.BlockSpec((1000, IN_DIM), lambda i, c: (i, 0)),
            pl.BlockSpec((1000, 1), lambda i, c: (i, 0)),
            pl.BlockSpec((IN_DIM, HIDDEN // 2), lambda i, c: (0, c)),
        ],
        out_specs=pl.BlockSpec((1, 1000, HIDDEN // 2), lambda i, c: (c, i, 0)),
        out_shape=jax.ShapeDtypeStruct((2, N, HIDDEN // 2), _f32),
    )(x, deg2d, W1)


def _mm2_body(agg_ref, deg_ref, b1_ref, w_ref, out_ref):
    sc = lax.rsqrt(deg_ref[...])
    acat = jnp.concatenate([agg_ref[0], agg_ref[1]], axis=1)
    u = sc * jax.nn.relu(sc * acat + b1_ref[...])
    out_ref[0] = jnp.dot(u, w_ref[...],
                         preferred_element_type=_f32,
                         precision=lax.Precision.HIGHEST)


def _mm2(agg1, deg2d, b1, W2):
    grid = (N // 1000, 2)
    return pl.pallas_call(
        _mm2_body,
        grid=grid,
        in_specs=[
            pl.BlockSpec((2, 1000, HIDDEN // 2), lambda i, c: (0, i, 0)),
            pl.BlockSpec((1000, 1), lambda i, c: (i, 0)),
            pl.BlockSpec((1, HIDDEN), lambda i, c: (0, 0)),
            pl.BlockSpec((HIDDEN, OUT_DIM // 2), lambda i, c: (0, c)),
        ],
        out_specs=pl.BlockSpec((1, 1000, OUT_DIM // 2), lambda i, c: (c, i, 0)),
        out_shape=jax.ShapeDtypeStruct((2, N, OUT_DIM // 2), _f32),
    )(agg1, deg2d, b1, W2)


def _pool_body(agg_ref, deg_ref, b2_ref, brow_ref, bcol_ref,
               z_ref, zg_ref, sums_scr, cnt_scr, mx_scr):
    i = pl.program_id(0)
    nblk = pl.num_programs(0)
    sc = lax.rsqrt(deg_ref[...])
    acat = jnp.concatenate([agg_ref[0], agg_ref[1]], axis=1)
    z = sc * acat + b2_ref[...]
    z_ref[...] = z

    @pl.when(i == 0)
    def _():
        sums_scr[...] = jnp.zeros_like(sums_scr)
        cnt_scr[...] = jnp.zeros_like(cnt_scr)
        mx_scr[...] = jnp.full_like(mx_scr, -jnp.inf)

    brow = brow_ref[0]
    seg_ids = lax.broadcasted_iota(jnp.int32, (B, 1), 0)
    onehot = (brow == seg_ids).astype(_f32)
    sums_scr[...] += jnp.dot(onehot, z, preferred_element_type=_f32,
                             precision=lax.Precision.HIGHEST)
    cnt_scr[...] += jnp.sum(onehot, axis=1, keepdims=True)

    bcol = bcol_ref[0]
    for b in range(B):
        masked = jnp.where(bcol == b, z, -jnp.inf)
        row = jnp.max(masked, axis=0, keepdims=True)
        mx_scr[pl.ds(b, 1), :] = jnp.maximum(mx_scr[pl.ds(b, 1), :], row)

    @pl.when(i == nblk - 1)
    def _():
        mean = sums_scr[...] / jnp.maximum(cnt_scr[...], 1.0)
        zg_ref[:, :OUT_DIM] = mean
        zg_ref[:, OUT_DIM:] = mx_scr[...]


def _pool(agg2, deg2d, b2, brow3, bcol3):
    grid = (N // 1000,)
    return pl.pallas_call(
        _pool_body,
        grid=grid,
        in_specs=[
            pl.BlockSpec((2, 1000, OUT_DIM // 2), lambda i: (0, i, 0)),
            pl.BlockSpec((1000, 1), lambda i: (i, 0)),
            pl.BlockSpec((1, OUT_DIM), lambda i: (0, 0)),
            pl.BlockSpec((1, 1, 1000), lambda i: (i, 0, 0)),
            pl.BlockSpec((1, 1000, 1), lambda i: (i, 0, 0)),
        ],
        out_specs=[
            pl.BlockSpec((1000, OUT_DIM), lambda i: (i, 0)),
            pl.BlockSpec((B, 2 * OUT_DIM), lambda i: (0, 0)),
        ],
        out_shape=[
            jax.ShapeDtypeStruct((N, OUT_DIM), _f32),
            jax.ShapeDtypeStruct((B, 2 * OUT_DIM), _f32),
        ],
        scratch_shapes=[
            pltpu.VMEM((B, OUT_DIM), _f32),
            pltpu.VMEM((B, 1), _f32),
            pltpu.VMEM((B, OUT_DIM), _f32),
        ],
    )(agg2, deg2d, b2, brow3, bcol3)


def kernel(x, edge_index, batch, W1, b1, W2, b2):
    src = edge_index[0]
    dst = edge_index[1]
    src_g = src.reshape(NSUB, NCH, K)
    dst_g = dst.reshape(NSUB, NCH, K)
    dst_d = dst.reshape(NSUB, DEG_NCH, DEG_CHUNK)

    deg = _degrees(dst_d)
    deg2d = deg.reshape(N, 1)

    hs1 = _mm1(x, deg2d, W1)
    agg1 = _edge_aggregate(hs1, src_g, dst_g, HIDDEN // 2)
    hs2 = _mm2(agg1, deg2d, b1.reshape(1, HIDDEN), W2)
    agg2 = _edge_aggregate(hs2, src_g, dst_g, OUT_DIM // 2)

    brow3 = batch.reshape(N // 1000, 1, 1000)
    bcol3 = batch.reshape(N // 1000, 1000, 1)
    z, z_g = _pool(agg2, deg2d, b2.reshape(1, OUT_DIM), brow3, bcol3)
    return (z, z_g)


# speedup vs baseline: 15.4944x; 15.4944x over previous
import functools

import jax
import jax.numpy as jnp
from jax import lax
from jax.experimental import pallas as pl
from jax.experimental.pallas import tpu as pltpu
from jax.experimental.pallas import tpu_sc as plsc

N = 10000
E = 320000
IN_DIM = 128
HIDDEN = 256
OUT_DIM = 128
B = 64

NSUB = 16
K = 256
NCH = 80
E_PAD = NSUB * NCH * K
EDGES_PER_SUB = NCH * K
DEG_CHUNK = 5120
DEG_NCH = EDGES_PER_SUB // DEG_CHUNK
SINK = N
N_ACC = N + 16
ROWS_PER_IO_SUB = 1000

_f32 = jnp.float32


def _vector_mesh():
    return plsc.VectorSubcoreMesh(core_axis_name="c", subcore_axis_name="s")


def _deg_body(dst_hbm, deg_hbm, ones_v, idx_v, stage_v, acc_sh, sem):
    c = lax.axis_index("c")
    s = lax.axis_index("s")
    ones_v[...] = jnp.ones((DEG_CHUNK,), _f32)

    @pl.when((c == 0) & (s < 10))
    def _():
        pltpu.sync_copy(ones_v.at[pl.ds(0, ROWS_PER_IO_SUB)],
                        acc_sh.at[pl.ds(s * ROWS_PER_IO_SUB, ROWS_PER_IO_SUB)])

    plsc.subcore_barrier()

    @pl.when(c == 0)
    def _():
        @pl.loop(0, DEG_NCH)
        def _(j):
            pltpu.sync_copy(dst_hbm.at[s].at[j], idx_v)
            pltpu.sync_copy(ones_v, acc_sh.at[idx_v], add=True)

    plsc.subcore_barrier()

    @pl.when((c == 0) & (s < 10))
    def _():
        rows = pl.ds(s * ROWS_PER_IO_SUB, ROWS_PER_IO_SUB)
        pltpu.sync_copy(acc_sh.at[rows], stage_v)
        pltpu.sync_copy(stage_v, deg_hbm.at[rows])


def _degrees(dst_d):
    kern = pl.kernel(
        _deg_body,
        out_type=jax.ShapeDtypeStruct((N,), _f32),
        mesh=_vector_mesh(),
        scratch_types=[
            pltpu.VMEM((DEG_CHUNK,), _f32),
            pltpu.VMEM((DEG_CHUNK,), jnp.int32),
            pltpu.VMEM((ROWS_PER_IO_SUB,), _f32),
            pltpu.VMEM_SHARED((N_ACC,), _f32),
            pltpu.SemaphoreType.DMA,
        ],
    )
    return kern(dst_d)


def _scatter_body(half, hs_hbm, src_hbm, dst_hbm, agg_hbm,
                  src_v, dst_v, data_v, acc_sh, sem):
    c = lax.axis_index("c")
    s = lax.axis_index("s")
    table = hs_hbm.at[c]

    @pl.when(s < 10)
    def _():
        rows = pl.ds(s * ROWS_PER_IO_SUB, ROWS_PER_IO_SUB)
        pltpu.sync_copy(table.at[rows], acc_sh.at[rows])

    plsc.subcore_barrier()

    @pl.loop(0, NCH)
    def _(j):
        pltpu.sync_copy(src_hbm.at[s].at[j], src_v)
        pltpu.sync_copy(dst_hbm.at[s].at[j], dst_v)
        pltpu.async_copy(table.at[src_v], data_v, sem).wait()
        pltpu.sync_copy(data_v, acc_sh.at[dst_v], add=True)

    plsc.subcore_barrier()

    @pl.when(s < 10)
    def _():
        rows = pl.ds(s * ROWS_PER_IO_SUB, ROWS_PER_IO_SUB)
        pltpu.sync_copy(acc_sh.at[rows], agg_hbm.at[c].at[rows])


def _edge_aggregate(hs, src_g, dst_g, half):
    kern = pl.kernel(
        functools.partial(_scatter_body, half),
        out_type=jax.ShapeDtypeStruct((2, N, half), _f32),
        mesh=_vector_mesh(),
        scratch_types=[
            pltpu.VMEM((K,), jnp.int32),
            pltpu.VMEM((K,), jnp.int32),
            pltpu.VMEM((K, half), _f32),
            pltpu.VMEM_SHARED((N_ACC, half), _f32),
            pltpu.SemaphoreType.DMA,
        ],
    )
    return kern(hs, src_g, dst_g)


NCH2 = E_PAD // (2 * NSUB * K)


def _scatter2_body(hs_hbm, src_hbm, dst_hbm, agg_hbm,
                   src_v, dst_v, data_v, acc_sh, sem):
    c = lax.axis_index("c")
    s = lax.axis_index("s")
    w = c * NSUB + s

    @pl.when(s < 10)
    def _():
        rows = pl.ds(s * ROWS_PER_IO_SUB, ROWS_PER_IO_SUB)

        @pl.when(c == 0)
        def _():
            pltpu.sync_copy(hs_hbm.at[rows], acc_sh.at[rows])

        @pl.when(c == 1)
        def _():
            data_v[...] = jnp.zeros((K, OUT_DIM), _f32)
            @pl.loop(0, 4)
            def _(k):
                pltpu.sync_copy(
                    data_v.at[pl.ds(0, 250)],
                    acc_sh.at[pl.ds(s * ROWS_PER_IO_SUB + k * 250, 250)])

    plsc.subcore_barrier()

    @pl.loop(0, NCH2)
    def _(j):
        pltpu.sync_copy(src_hbm.at[w].at[j], src_v)
        pltpu.sync_copy(dst_hbm.at[w].at[j], dst_v)
        pltpu.async_copy(hs_hbm.at[src_v], data_v, sem).wait()
        pltpu.sync_copy(data_v, acc_sh.at[dst_v], add=True)

    plsc.subcore_barrier()

    @pl.when(s < 10)
    def _():
        rows = pl.ds(s * ROWS_PER_IO_SUB, ROWS_PER_IO_SUB)
        pltpu.sync_copy(acc_sh.at[rows], agg_hbm.at[c].at[rows])


def _edge_aggregate2(hs, src_g2, dst_g2):
    kern = pl.kernel(
        _scatter2_body,
        out_type=jax.ShapeDtypeStruct((2, N, OUT_DIM), _f32),
        mesh=_vector_mesh(),
        scratch_types=[
            pltpu.VMEM((K,), jnp.int32),
            pltpu.VMEM((K,), jnp.int32),
            pltpu.VMEM((K, OUT_DIM), _f32),
            pltpu.VMEM_SHARED((N_ACC, OUT_DIM), _f32),
            pltpu.SemaphoreType.DMA,
        ],
    )
    return kern(hs, src_g2, dst_g2)


def _mm1_body(x_ref, deg_ref, w_ref, out_ref):
    sc = lax.rsqrt(deg_ref[...])
    out_ref[0] = jnp.dot(x_ref[...] * sc, w_ref[...],
                         preferred_element_type=_f32,
                         precision=lax.Precision.HIGHEST)


def _mm1(x, deg2d, W1):
    grid = (N // 1000, 2)
    return pl.pallas_call(
        _mm1_body,
        grid=grid,
        in_specs=[
            pl.BlockSpec((1000, IN_DIM), lambda i, c: (i, 0)),
            pl.BlockSpec((1000, 1), lambda i, c: (i, 0)),
            pl.BlockSpec((IN_DIM, HIDDEN // 2), lambda i, c: (0, c)),
        ],
        out_specs=pl.BlockSpec((1, 1000, HIDDEN // 2), lambda i, c: (c, i, 0)),
        out_shape=jax.ShapeDtypeStruct((2, N, HIDDEN // 2), _f32),
    )(x, deg2d, W1)


def _mm2_body(agg_ref, deg_ref, b1_ref, w_ref, out_ref):
    sc = lax.rsqrt(deg_ref[...])
    acat = jnp.concatenate([agg_ref[0], agg_ref[1]], axis=1)
    u = sc * jax.nn.relu(sc * acat + b1_ref[...])
    out_ref[...] = jnp.dot(u, w_ref[...],
                           preferred_element_type=_f32,
                           precision=lax.Precision.HIGHEST)


def _mm2(agg1, deg2d, b1, W2):
    grid = (N // 1000,)
    return pl.pallas_call(
        _mm2_body,
        grid=grid,
        in_specs=[
            pl.BlockSpec((2, 1000, HIDDEN // 2), lambda i: (0, i, 0)),
            pl.BlockSpec((1000, 1), lambda i: (i, 0)),
            pl.BlockSpec((1, HIDDEN), lambda i: (0, 0)),
            pl.BlockSpec((HIDDEN, OUT_DIM), lambda i: (0, 0)),
        ],
        out_specs=pl.BlockSpec((1000, OUT_DIM), lambda i: (i, 0)),
        out_shape=jax.ShapeDtypeStruct((N, OUT_DIM), _f32),
    )(agg1, deg2d, b1, W2)


def _pool_body(agg_ref, deg_ref, b2_ref, brow_ref, bcol_ref,
               z_ref, zg_ref, sums_scr, cnt_scr, mx_scr):
    i = pl.program_id(0)
    nblk = pl.num_programs(0)
    sc = lax.rsqrt(deg_ref[...])
    acat = agg_ref[0] + agg_ref[1]
    z = sc * acat + b2_ref[...]
    z_ref[...] = z

    @pl.when(i == 0)
    def _():
        sums_scr[...] = jnp.zeros_like(sums_scr)
        cnt_scr[...] = jnp.zeros_like(cnt_scr)
        mx_scr[...] = jnp.full_like(mx_scr, -jnp.inf)

    brow = brow_ref[0]
    seg_ids = lax.broadcasted_iota(jnp.int32, (B, 1), 0)
    onehot = (brow == seg_ids).astype(_f32)
    sums_scr[...] += jnp.dot(onehot, z, preferred_element_type=_f32,
                             precision=lax.Precision.HIGHEST)
    cnt_scr[...] += jnp.sum(onehot, axis=1, keepdims=True)

    bcol = bcol_ref[0]
    for b in range(B):
        masked = jnp.where(bcol == b, z, -jnp.inf)
        row = jnp.max(masked, axis=0, keepdims=True)
        mx_scr[pl.ds(b, 1), :] = jnp.maximum(mx_scr[pl.ds(b, 1), :], row)

    @pl.when(i == nblk - 1)
    def _():
        mean = sums_scr[...] / jnp.maximum(cnt_scr[...], 1.0)
        zg_ref[:, :OUT_DIM] = mean
        zg_ref[:, OUT_DIM:] = mx_scr[...]


def _pool(agg2, deg2d, b2, brow3, bcol3):
    grid = (N // 1000,)
    return pl.pallas_call(
        _pool_body,
        grid=grid,
        in_specs=[
            pl.BlockSpec((2, 1000, OUT_DIM), lambda i: (0, i, 0)),
            pl.BlockSpec((1000, 1), lambda i: (i, 0)),
            pl.BlockSpec((1, OUT_DIM), lambda i: (0, 0)),
            pl.BlockSpec((1, 1, 1000), lambda i: (i, 0, 0)),
            pl.BlockSpec((1, 1000, 1), lambda i: (i, 0, 0)),
        ],
        out_specs=[
            pl.BlockSpec((1000, OUT_DIM), lambda i: (i, 0)),
            pl.BlockSpec((B, 2 * OUT_DIM), lambda i: (0, 0)),
        ],
        out_shape=[
            jax.ShapeDtypeStruct((N, OUT_DIM), _f32),
            jax.ShapeDtypeStruct((B, 2 * OUT_DIM), _f32),
        ],
        scratch_shapes=[
            pltpu.VMEM((B, OUT_DIM), _f32),
            pltpu.VMEM((B, 1), _f32),
            pltpu.VMEM((B, OUT_DIM), _f32),
        ],
    )(agg2, deg2d, b2, brow3, bcol3)


def kernel(x, edge_index, batch, W1, b1, W2, b2):
    pad = E_PAD - E
    pad_src = jnp.arange(pad, dtype=jnp.int32) % N
    pad_dst = SINK + (jnp.arange(pad, dtype=jnp.int32) % (N_ACC - SINK))
    src = jnp.concatenate([edge_index[0], pad_src])
    dst = jnp.concatenate([edge_index[1], pad_dst])
    src_g = src.reshape(NSUB, NCH, K)
    dst_g = dst.reshape(NSUB, NCH, K)
    src_g2 = src.reshape(2 * NSUB, NCH2, K)
    dst_g2 = dst.reshape(2 * NSUB, NCH2, K)
    dst_d = dst.reshape(NSUB, DEG_NCH, DEG_CHUNK)

    deg = _degrees(dst_d)
    deg2d = deg.reshape(N, 1)

    hs1 = _mm1(x, deg2d, W1)
    agg1 = _edge_aggregate(hs1, src_g, dst_g, HIDDEN // 2)
    hs2 = _mm2(agg1, deg2d, b1.reshape(1, HIDDEN), W2)
    agg2 = _edge_aggregate2(hs2, src_g2, dst_g2)

    brow3 = batch.reshape(N // 1000, 1, 1000)
    bcol3 = batch.reshape(N // 1000, 1000, 1)
    z, z_g = _pool(agg2, deg2d, b2.reshape(1, OUT_DIM), brow3, bcol3)
    return (z, z_g)


# speedup vs baseline: 18.2885x; 1.1803x over previous
import functools

import jax
import jax.numpy as jnp
from jax import lax
from jax.experimental import pallas as pl
from jax.experimental.pallas import tpu as pltpu
from jax.experimental.pallas import tpu_sc as plsc

N = 10000
E = 320000
IN_DIM = 128
HIDDEN = 256
OUT_DIM = 128
B = 64

NSUB = 16
K = 128
NCH = 160
E_PAD = NSUB * NCH * K
EDGES_PER_SUB = NCH * K
DEG_CHUNK = 5120
DEG_NCH = EDGES_PER_SUB // DEG_CHUNK
SINK = N
N_ACC = N + 16
ROWS_PER_IO_SUB = 1000

_f32 = jnp.float32


def _vector_mesh():
    return plsc.VectorSubcoreMesh(core_axis_name="c", subcore_axis_name="s")


def _deg_body(dst_hbm, deg_hbm, ones_v, idx_v, stage_v, acc_sh, sem):
    c = lax.axis_index("c")
    s = lax.axis_index("s")
    ones_v[...] = jnp.ones((DEG_CHUNK,), _f32)

    @pl.when((c == 0) & (s < 10))
    def _():
        pltpu.sync_copy(ones_v.at[pl.ds(0, ROWS_PER_IO_SUB)],
                        acc_sh.at[pl.ds(s * ROWS_PER_IO_SUB, ROWS_PER_IO_SUB)])

    plsc.subcore_barrier()

    @pl.when(c == 0)
    def _():
        @pl.loop(0, DEG_NCH)
        def _(j):
            pltpu.sync_copy(dst_hbm.at[s].at[j], idx_v)
            pltpu.sync_copy(ones_v, acc_sh.at[idx_v], add=True)

    plsc.subcore_barrier()

    @pl.when((c == 0) & (s < 10))
    def _():
        rows = pl.ds(s * ROWS_PER_IO_SUB, ROWS_PER_IO_SUB)
        pltpu.sync_copy(acc_sh.at[rows], stage_v)
        pltpu.sync_copy(stage_v, deg_hbm.at[rows])


def _degrees(dst_d):
    kern = pl.kernel(
        _deg_body,
        out_type=jax.ShapeDtypeStruct((N,), _f32),
        mesh=_vector_mesh(),
        scratch_types=[
            pltpu.VMEM((DEG_CHUNK,), _f32),
            pltpu.VMEM((DEG_CHUNK,), jnp.int32),
            pltpu.VMEM((ROWS_PER_IO_SUB,), _f32),
            pltpu.VMEM_SHARED((N_ACC,), _f32),
            pltpu.SemaphoreType.DMA,
        ],
    )
    return kern(dst_d)


def _pipelined_gather_scatter(table, src_rows, dst_rows, nch, acc_sh,
                              srcv, dstv, datav, sems):
    def load_idx(j, b):
        pltpu.sync_copy(src_rows.at[j], srcv[b])
        pltpu.sync_copy(dst_rows.at[j], dstv[b])

    def gather(b):
        return pltpu.make_async_copy(table.at[srcv[b]], datav[b], sems[b])

    load_idx(0, 0)
    gather(0).start()
    load_idx(1, 1)
    gather(1).start()
    plsc.subcore_barrier()

    @pl.loop(0, nch, step=2)
    def _(j):
        for b in range(2):
            jj = j + b
            gather(b).wait()
            pltpu.sync_copy(datav[b], acc_sh.at[dstv[b]], add=True)

            @pl.when(jj + 2 < nch)
            def _():
                load_idx(jj + 2, b)
                gather(b).start()

    plsc.subcore_barrier()


def _scatter_body(half, hs_hbm, src_hbm, dst_hbm, agg_hbm,
                  src_v0, src_v1, dst_v0, dst_v1, data_v0, data_v1,
                  acc_sh, sem0, sem1):
    c = lax.axis_index("c")
    s = lax.axis_index("s")
    table = hs_hbm.at[c]

    @pl.when(s < 10)
    def _():
        rows = pl.ds(s * ROWS_PER_IO_SUB, ROWS_PER_IO_SUB)
        pltpu.sync_copy(table.at[rows], acc_sh.at[rows])

    _pipelined_gather_scatter(table, src_hbm.at[s], dst_hbm.at[s], NCH, acc_sh,
                              (src_v0, src_v1), (dst_v0, dst_v1),
                              (data_v0, data_v1), (sem0, sem1))

    @pl.when(s < 10)
    def _():
        rows = pl.ds(s * ROWS_PER_IO_SUB, ROWS_PER_IO_SUB)
        pltpu.sync_copy(acc_sh.at[rows], agg_hbm.at[c].at[rows])


def _edge_aggregate(hs, src_g, dst_g, half):
    kern = pl.kernel(
        functools.partial(_scatter_body, half),
        out_type=jax.ShapeDtypeStruct((2, N, half), _f32),
        mesh=_vector_mesh(),
        scratch_types=[
            pltpu.VMEM((K,), jnp.int32),
            pltpu.VMEM((K,), jnp.int32),
            pltpu.VMEM((K,), jnp.int32),
            pltpu.VMEM((K,), jnp.int32),
            pltpu.VMEM((K, half), _f32),
            pltpu.VMEM((K, half), _f32),
            pltpu.VMEM_SHARED((N_ACC, half), _f32),
            pltpu.SemaphoreType.DMA,
            pltpu.SemaphoreType.DMA,
        ],
    )
    return kern(hs, src_g, dst_g)


NCH2 = E_PAD // (2 * NSUB * K)


def _scatter2_body(hs_hbm, src_hbm, dst_hbm, agg_hbm,
                   src_v0, src_v1, dst_v0, dst_v1, data_v0, data_v1,
                   acc_sh, sem0, sem1):
    c = lax.axis_index("c")
    s = lax.axis_index("s")
    w = c * NSUB + s

    @pl.when(s < 10)
    def _():
        rows = pl.ds(s * ROWS_PER_IO_SUB, ROWS_PER_IO_SUB)

        @pl.when(c == 0)
        def _():
            pltpu.sync_copy(hs_hbm.at[rows], acc_sh.at[rows])

        @pl.when(c == 1)
        def _():
            data_v0[...] = jnp.zeros((K, OUT_DIM), _f32)
            @pl.loop(0, 8)
            def _(k):
                pltpu.sync_copy(
                    data_v0.at[pl.ds(0, 125)],
                    acc_sh.at[pl.ds(s * ROWS_PER_IO_SUB + k * 125, 125)])

    _pipelined_gather_scatter(hs_hbm, src_hbm.at[w], dst_hbm.at[w], NCH2,
                              acc_sh, (src_v0, src_v1), (dst_v0, dst_v1),
                              (data_v0, data_v1), (sem0, sem1))

    @pl.when(s < 10)
    def _():
        rows = pl.ds(s * ROWS_PER_IO_SUB, ROWS_PER_IO_SUB)
        pltpu.sync_copy(acc_sh.at[rows], agg_hbm.at[c].at[rows])


def _edge_aggregate2(hs, src_g2, dst_g2):
    kern = pl.kernel(
        _scatter2_body,
        out_type=jax.ShapeDtypeStruct((2, N, OUT_DIM), _f32),
        mesh=_vector_mesh(),
        scratch_types=[
            pltpu.VMEM((K,), jnp.int32),
            pltpu.VMEM((K,), jnp.int32),
            pltpu.VMEM((K,), jnp.int32),
            pltpu.VMEM((K,), jnp.int32),
            pltpu.VMEM((K, OUT_DIM), _f32),
            pltpu.VMEM((K, OUT_DIM), _f32),
            pltpu.VMEM_SHARED((N_ACC, OUT_DIM), _f32),
            pltpu.SemaphoreType.DMA,
            pltpu.SemaphoreType.DMA,
        ],
    )
    return kern(hs, src_g2, dst_g2)


def _mm1_body(x_ref, deg_ref, w_ref, out_ref):
    sc = lax.rsqrt(deg_ref[...])
    out_ref[0] = jnp.dot(x_ref[...] * sc, w_ref[...],
                         preferred_element_type=_f32,
                         precision=lax.Precision.HIGHEST)


def _mm1(x, deg2d, W1):
    grid = (N // 1000, 2)
    return pl.pallas_call(
        _mm1_body,
        grid=grid,
        in_specs=[
            pl.BlockSpec((1000, IN_DIM), lambda i, c: (i, 0)),
            pl.BlockSpec((1000, 1), lambda i, c: (i, 0)),
            pl.BlockSpec((IN_DIM, HIDDEN // 2), lambda i, c: (0, c)),
        ],
        out_specs=pl.BlockSpec((1, 1000, HIDDEN // 2), lambda i, c: (c, i, 0)),
        out_shape=jax.ShapeDtypeStruct((2, N, HIDDEN // 2), _f32),
    )(x, deg2d, W1)


def _mm2_body(agg_ref, deg_ref, b1_ref, w_ref, out_ref):
    sc = lax.rsqrt(deg_ref[...])
    acat = jnp.concatenate([agg_ref[0], agg_ref[1]], axis=1)
    u = sc * jax.nn.relu(sc * acat + b1_ref[...])
    out_ref[...] = jnp.dot(u, w_ref[...],
                           preferred_element_type=_f32,
                           precision=lax.Precision.HIGHEST)


def _mm2(agg1, deg2d, b1, W2):
    grid = (N // 1000,)
    return pl.pallas_call(
        _mm2_body,
        grid=grid,
        in_specs=[
            pl.BlockSpec((2, 1000, HIDDEN // 2), lambda i: (0, i, 0)),
            pl.BlockSpec((1000, 1), lambda i: (i, 0)),
            pl.BlockSpec((1, HIDDEN), lambda i: (0, 0)),
            pl.BlockSpec((HIDDEN, OUT_DIM), lambda i: (0, 0)),
        ],
        out_specs=pl.BlockSpec((1000, OUT_DIM), lambda i: (i, 0)),
        out_shape=jax.ShapeDtypeStruct((N, OUT_DIM), _f32),
    )(agg1, deg2d, b1, W2)


def _pool_body(agg_ref, deg_ref, b2_ref, brow_ref, bcol_ref,
               z_ref, zg_ref, sums_scr, cnt_scr, mx_scr):
    i = pl.program_id(0)
    nblk = pl.num_programs(0)
    sc = lax.rsqrt(deg_ref[...])
    acat = agg_ref[0] + agg_ref[1]
    z = sc * acat + b2_ref[...]
    z_ref[...] = z

    @pl.when(i == 0)
    def _():
        sums_scr[...] = jnp.zeros_like(sums_scr)
        cnt_scr[...] = jnp.zeros_like(cnt_scr)
        mx_scr[...] = jnp.full_like(mx_scr, -jnp.inf)

    brow = brow_ref[0]
    seg_ids = lax.broadcasted_iota(jnp.int32, (B, 1), 0)
    onehot = (brow == seg_ids).astype(_f32)
    sums_scr[...] += jnp.dot(onehot, z, preferred_element_type=_f32,
                             precision=lax.Precision.HIGHEST)
    cnt_scr[...] += jnp.sum(onehot, axis=1, keepdims=True)

    bcol = bcol_ref[0]
    for b in range(B):
        masked = jnp.where(bcol == b, z, -jnp.inf)
        row = jnp.max(masked, axis=0, keepdims=True)
        mx_scr[pl.ds(b, 1), :] = jnp.maximum(mx_scr[pl.ds(b, 1), :], row)

    @pl.when(i == nblk - 1)
    def _():
        mean = sums_scr[...] / jnp.maximum(cnt_scr[...], 1.0)
        zg_ref[:, :OUT_DIM] = mean
        zg_ref[:, OUT_DIM:] = mx_scr[...]


def _pool(agg2, deg2d, b2, brow3, bcol3):
    grid = (N // 1000,)
    return pl.pallas_call(
        _pool_body,
        grid=grid,
        in_specs=[
            pl.BlockSpec((2, 1000, OUT_DIM), lambda i: (0, i, 0)),
            pl.BlockSpec((1000, 1), lambda i: (i, 0)),
            pl.BlockSpec((1, OUT_DIM), lambda i: (0, 0)),
            pl.BlockSpec((1, 1, 1000), lambda i: (i, 0, 0)),
            pl.BlockSpec((1, 1000, 1), lambda i: (i, 0, 0)),
        ],
        out_specs=[
            pl.BlockSpec((1000, OUT_DIM), lambda i: (i, 0)),
            pl.BlockSpec((B, 2 * OUT_DIM), lambda i: (0, 0)),
        ],
        out_shape=[
            jax.ShapeDtypeStruct((N, OUT_DIM), _f32),
            jax.ShapeDtypeStruct((B, 2 * OUT_DIM), _f32),
        ],
        scratch_shapes=[
            pltpu.VMEM((B, OUT_DIM), _f32),
            pltpu.VMEM((B, 1), _f32),
            pltpu.VMEM((B, OUT_DIM), _f32),
        ],
    )(agg2, deg2d, b2, brow3, bcol3)


def kernel(x, edge_index, batch, W1, b1, W2, b2):
    pad = E_PAD - E
    pad_src = jnp.arange(pad, dtype=jnp.int32) % N
    pad_dst = SINK + (jnp.arange(pad, dtype=jnp.int32) % (N_ACC - SINK))
    src = jnp.concatenate([edge_index[0], pad_src])
    dst = jnp.concatenate([edge_index[1], pad_dst])
    src_g = src.reshape(NSUB, NCH, K)
    dst_g = dst.reshape(NSUB, NCH, K)
    src_g2 = src.reshape(2 * NSUB, NCH2, K)
    dst_g2 = dst.reshape(2 * NSUB, NCH2, K)
    dst_d = dst.reshape(NSUB, DEG_NCH, DEG_CHUNK)

    deg = _degrees(dst_d)
    deg2d = deg.reshape(N, 1)

    hs1 = _mm1(x, deg2d, W1)
    agg1 = _edge_aggregate(hs1, src_g, dst_g, HIDDEN // 2)
    hs2 = _mm2(agg1, deg2d, b1.reshape(1, HIDDEN), W2)
    agg2 = _edge_aggregate2(hs2, src_g2, dst_g2)

    brow3 = batch.reshape(N // 1000, 1, 1000)
    bcol3 = batch.reshape(N // 1000, 1000, 1)
    z, z_g = _pool(agg2, deg2d, b2.reshape(1, OUT_DIM), brow3, bcol3)
    return (z, z_g)


# speedup vs baseline: 20.2143x; 1.1053x over previous
import functools

import jax
import jax.numpy as jnp
from jax import lax
from jax.experimental import pallas as pl
from jax.experimental.pallas import tpu as pltpu
from jax.experimental.pallas import tpu_sc as plsc

N = 10000
E = 320000
IN_DIM = 128
HIDDEN = 256
OUT_DIM = 128
B = 64

NSUB = 16
K = 128
NCH = 160
E_PAD = NSUB * NCH * K
EDGES_PER_SUB = NCH * K
DEG_CHUNK = 5120
DEG_NCH = EDGES_PER_SUB // DEG_CHUNK
SINK = N
N_ACC = N + 16
ROWS_PER_IO_SUB = 1000

_f32 = jnp.float32


def _vector_mesh():
    return plsc.VectorSubcoreMesh(core_axis_name="c", subcore_axis_name="s")


def _deg_body(dst_hbm, deg_hbm, ones_v, idx_v, stage_v, acc_sh, sem):
    c = lax.axis_index("c")
    s = lax.axis_index("s")
    ones_v[...] = jnp.ones((DEG_CHUNK,), _f32)

    @pl.when((c == 0) & (s < 10))
    def _():
        pltpu.sync_copy(ones_v.at[pl.ds(0, ROWS_PER_IO_SUB)],
                        acc_sh.at[pl.ds(s * ROWS_PER_IO_SUB, ROWS_PER_IO_SUB)])

    plsc.subcore_barrier()

    @pl.when(c == 0)
    def _():
        @pl.loop(0, DEG_NCH)
        def _(j):
            pltpu.sync_copy(dst_hbm.at[s].at[j], idx_v)
            pltpu.sync_copy(ones_v, acc_sh.at[idx_v], add=True)

    plsc.subcore_barrier()

    @pl.when((c == 0) & (s < 10))
    def _():
        rows = pl.ds(s * ROWS_PER_IO_SUB, ROWS_PER_IO_SUB)
        pltpu.sync_copy(acc_sh.at[rows], stage_v)
        pltpu.sync_copy(stage_v, deg_hbm.at[rows])


def _degrees(dst_d):
    kern = pl.kernel(
        _deg_body,
        out_type=jax.ShapeDtypeStruct((N,), _f32),
        mesh=_vector_mesh(),
        scratch_types=[
            pltpu.VMEM((DEG_CHUNK,), _f32),
            pltpu.VMEM((DEG_CHUNK,), jnp.int32),
            pltpu.VMEM((ROWS_PER_IO_SUB,), _f32),
            pltpu.VMEM_SHARED((N_ACC,), _f32),
            pltpu.SemaphoreType.DMA,
        ],
    )
    return kern(dst_d)


def _pipelined_gather_scatter(table, src_rows, dst_rows, nch, acc_sh,
                              srcv, dstv, datav, gsems, isems):
    def idx_copies(j, b):
        return (pltpu.make_async_copy(src_rows.at[j], srcv[b], isems[b]),
                pltpu.make_async_copy(dst_rows.at[j], dstv[b], isems[b]))

    def start_idx(j, b):
        for cp in idx_copies(j, b):
            cp.start()

    def wait_idx(j, b):
        for cp in idx_copies(j, b):
            cp.wait()

    def gather(b):
        return pltpu.make_async_copy(table.at[srcv[b]], datav[b], gsems[b])

    start_idx(0, 0)
    start_idx(1, 1)
    wait_idx(0, 0)
    gather(0).start()
    plsc.subcore_barrier()

    @pl.loop(0, nch, step=2)
    def _(j):
        for b in range(2):
            jj = j + b
            nb = 1 - b
            gather(b).wait()

            @pl.when(jj + 1 < nch)
            def _():
                wait_idx(jj + 1, nb)
                gather(nb).start()

            pltpu.sync_copy(datav[b], acc_sh.at[dstv[b]], add=True)

            @pl.when(jj + 2 < nch)
            def _():
                start_idx(jj + 2, b)

    plsc.subcore_barrier()


def _scatter_body(half, hs_hbm, src_hbm, dst_hbm, agg_hbm,
                  src_v0, src_v1, dst_v0, dst_v1, data_v0, data_v1,
                  acc_sh, sem0, sem1, semi0, semi1):
    c = lax.axis_index("c")
    s = lax.axis_index("s")
    table = hs_hbm.at[c]

    @pl.when(s < 10)
    def _():
        rows = pl.ds(s * ROWS_PER_IO_SUB, ROWS_PER_IO_SUB)
        pltpu.sync_copy(table.at[rows], acc_sh.at[rows])

    _pipelined_gather_scatter(table, src_hbm.at[s], dst_hbm.at[s], NCH, acc_sh,
                              (src_v0, src_v1), (dst_v0, dst_v1),
                              (data_v0, data_v1), (sem0, sem1), (semi0, semi1))

    @pl.when(s < 10)
    def _():
        rows = pl.ds(s * ROWS_PER_IO_SUB, ROWS_PER_IO_SUB)
        pltpu.sync_copy(acc_sh.at[rows], agg_hbm.at[c].at[rows])


def _edge_aggregate(hs, src_g, dst_g, half):
    kern = pl.kernel(
        functools.partial(_scatter_body, half),
        out_type=jax.ShapeDtypeStruct((2, N, half), _f32),
        mesh=_vector_mesh(),
        scratch_types=[
            pltpu.VMEM((K,), jnp.int32),
            pltpu.VMEM((K,), jnp.int32),
            pltpu.VMEM((K,), jnp.int32),
            pltpu.VMEM((K,), jnp.int32),
            pltpu.VMEM((K, half), _f32),
            pltpu.VMEM((K, half), _f32),
            pltpu.VMEM_SHARED((N_ACC, half), _f32),
            pltpu.SemaphoreType.DMA,
            pltpu.SemaphoreType.DMA,
            pltpu.SemaphoreType.DMA,
            pltpu.SemaphoreType.DMA,
        ],
    )
    return kern(hs, src_g, dst_g)


NCH2 = E_PAD // (2 * NSUB * K)


def _scatter2_body(hs_hbm, src_hbm, dst_hbm, agg_hbm,
                   src_v0, src_v1, dst_v0, dst_v1, data_v0, data_v1,
                   acc_sh, sem0, sem1, semi0, semi1):
    c = lax.axis_index("c")
    s = lax.axis_index("s")
    w = c * NSUB + s

    @pl.when(s < 10)
    def _():
        rows = pl.ds(s * ROWS_PER_IO_SUB, ROWS_PER_IO_SUB)

        @pl.when(c == 0)
        def _():
            pltpu.sync_copy(hs_hbm.at[rows], acc_sh.at[rows])

        @pl.when(c == 1)
        def _():
            data_v0[...] = jnp.zeros((K, OUT_DIM), _f32)
            @pl.loop(0, 8)
            def _(k):
                pltpu.sync_copy(
                    data_v0.at[pl.ds(0, 125)],
                    acc_sh.at[pl.ds(s * ROWS_PER_IO_SUB + k * 125, 125)])

    _pipelined_gather_scatter(hs_hbm, src_hbm.at[w], dst_hbm.at[w], NCH2,
                              acc_sh, (src_v0, src_v1), (dst_v0, dst_v1),
                              (data_v0, data_v1), (sem0, sem1), (semi0, semi1))

    @pl.when(s < 10)
    def _():
        rows = pl.ds(s * ROWS_PER_IO_SUB, ROWS_PER_IO_SUB)
        pltpu.sync_copy(acc_sh.at[rows], agg_hbm.at[c].at[rows])


def _edge_aggregate2(hs, src_g2, dst_g2):
    kern = pl.kernel(
        _scatter2_body,
        out_type=jax.ShapeDtypeStruct((2, N, OUT_DIM), _f32),
        mesh=_vector_mesh(),
        scratch_types=[
            pltpu.VMEM((K,), jnp.int32),
            pltpu.VMEM((K,), jnp.int32),
            pltpu.VMEM((K,), jnp.int32),
            pltpu.VMEM((K,), jnp.int32),
            pltpu.VMEM((K, OUT_DIM), _f32),
            pltpu.VMEM((K, OUT_DIM), _f32),
            pltpu.VMEM_SHARED((N_ACC, OUT_DIM), _f32),
            pltpu.SemaphoreType.DMA,
            pltpu.SemaphoreType.DMA,
            pltpu.SemaphoreType.DMA,
            pltpu.SemaphoreType.DMA,
        ],
    )
    return kern(hs, src_g2, dst_g2)


def _mm1_body(x_ref, deg_ref, w_ref, out_ref):
    sc = lax.rsqrt(deg_ref[...])
    out_ref[0] = jnp.dot(x_ref[...] * sc, w_ref[...],
                         preferred_element_type=_f32,
                         precision=lax.Precision.HIGHEST)


def _mm1(x, deg2d, W1):
    grid = (N // 1000, 2)
    return pl.pallas_call(
        _mm1_body,
        grid=grid,
        in_specs=[
            pl.BlockSpec((1000, IN_DIM), lambda i, c: (i, 0)),
            pl.BlockSpec((1000, 1), lambda i, c: (i, 0)),
            pl.BlockSpec((IN_DIM, HIDDEN // 2), lambda i, c: (0, c)),
        ],
        out_specs=pl.BlockSpec((1, 1000, HIDDEN // 2), lambda i, c: (c, i, 0)),
        out_shape=jax.ShapeDtypeStruct((2, N, HIDDEN // 2), _f32),
    )(x, deg2d, W1)


def _mm2_body(agg_ref, deg_ref, b1_ref, w_ref, out_ref):
    sc = lax.rsqrt(deg_ref[...])
    acat = jnp.concatenate([agg_ref[0], agg_ref[1]], axis=1)
    u = sc * jax.nn.relu(sc * acat + b1_ref[...])
    out_ref[...] = jnp.dot(u, w_ref[...],
                           preferred_element_type=_f32,
                           precision=lax.Precision.HIGHEST)


def _mm2(agg1, deg2d, b1, W2):
    grid = (N // 1000,)
    return pl.pallas_call(
        _mm2_body,
        grid=grid,
        in_specs=[
            pl.BlockSpec((2, 1000, HIDDEN // 2), lambda i: (0, i, 0)),
            pl.BlockSpec((1000, 1), lambda i: (i, 0)),
            pl.BlockSpec((1, HIDDEN), lambda i: (0, 0)),
            pl.BlockSpec((HIDDEN, OUT_DIM), lambda i: (0, 0)),
        ],
        out_specs=pl.BlockSpec((1000, OUT_DIM), lambda i: (i, 0)),
        out_shape=jax.ShapeDtypeStruct((N, OUT_DIM), _f32),
    )(agg1, deg2d, b1, W2)


def _pool_body(agg_ref, deg_ref, b2_ref, brow_ref, bcol_ref,
               z_ref, zg_ref, sums_scr, cnt_scr, mx_scr):
    i = pl.program_id(0)
    nblk = pl.num_programs(0)
    sc = lax.rsqrt(deg_ref[...])
    acat = agg_ref[0] + agg_ref[1]
    z = sc * acat + b2_ref[...]
    z_ref[...] = z

    @pl.when(i == 0)
    def _():
        sums_scr[...] = jnp.zeros_like(sums_scr)
        cnt_scr[...] = jnp.zeros_like(cnt_scr)
        mx_scr[...] = jnp.full_like(mx_scr, -jnp.inf)

    brow = brow_ref[0]
    seg_ids = lax.broadcasted_iota(jnp.int32, (B, 1), 0)
    onehot = (brow == seg_ids).astype(_f32)
    sums_scr[...] += jnp.dot(onehot, z, preferred_element_type=_f32,
                             precision=lax.Precision.HIGHEST)
    cnt_scr[...] += jnp.sum(onehot, axis=1, keepdims=True)

    bcol = bcol_ref[0]
    for b in range(B):
        masked = jnp.where(bcol == b, z, -jnp.inf)
        row = jnp.max(masked, axis=0, keepdims=True)
        mx_scr[pl.ds(b, 1), :] = jnp.maximum(mx_scr[pl.ds(b, 1), :], row)

    @pl.when(i == nblk - 1)
    def _():
        mean = sums_scr[...] / jnp.maximum(cnt_scr[...], 1.0)
        zg_ref[:, :OUT_DIM] = mean
        zg_ref[:, OUT_DIM:] = mx_scr[...]


def _pool(agg2, deg2d, b2, brow3, bcol3):
    grid = (N // 1000,)
    return pl.pallas_call(
        _pool_body,
        grid=grid,
        in_specs=[
            pl.BlockSpec((2, 1000, OUT_DIM), lambda i: (0, i, 0)),
            pl.BlockSpec((1000, 1), lambda i: (i, 0)),
            pl.BlockSpec((1, OUT_DIM), lambda i: (0, 0)),
            pl.BlockSpec((1, 1, 1000), lambda i: (i, 0, 0)),
            pl.BlockSpec((1, 1000, 1), lambda i: (i, 0, 0)),
        ],
        out_specs=[
            pl.BlockSpec((1000, OUT_DIM), lambda i: (i, 0)),
            pl.BlockSpec((B, 2 * OUT_DIM), lambda i: (0, 0)),
        ],
        out_shape=[
            jax.ShapeDtypeStruct((N, OUT_DIM), _f32),
            jax.ShapeDtypeStruct((B, 2 * OUT_DIM), _f32),
        ],
        scratch_shapes=[
            pltpu.VMEM((B, OUT_DIM), _f32),
            pltpu.VMEM((B, 1), _f32),
            pltpu.VMEM((B, OUT_DIM), _f32),
        ],
    )(agg2, deg2d, b2, brow3, bcol3)


def kernel(x, edge_index, batch, W1, b1, W2, b2):
    pad = E_PAD - E
    pad_src = jnp.arange(pad, dtype=jnp.int32) % N
    pad_dst = SINK + (jnp.arange(pad, dtype=jnp.int32) % (N_ACC - SINK))
    src = jnp.concatenate([edge_index[0], pad_src])
    dst = jnp.concatenate([edge_index[1], pad_dst])
    src_g = src.reshape(NSUB, NCH, K)
    dst_g = dst.reshape(NSUB, NCH, K)
    src_g2 = src.reshape(2 * NSUB, NCH2, K)
    dst_g2 = dst.reshape(2 * NSUB, NCH2, K)
    dst_d = dst.reshape(NSUB, DEG_NCH, DEG_CHUNK)

    deg = _degrees(dst_d)
    deg2d = deg.reshape(N, 1)

    hs1 = _mm1(x, deg2d, W1)
    agg1 = _edge_aggregate(hs1, src_g, dst_g, HIDDEN // 2)
    hs2 = _mm2(agg1, deg2d, b1.reshape(1, HIDDEN), W2)
    agg2 = _edge_aggregate2(hs2, src_g2, dst_g2)

    brow3 = batch.reshape(N // 1000, 1, 1000)
    bcol3 = batch.reshape(N // 1000, 1000, 1)
    z, z_g = _pool(agg2, deg2d, b2.reshape(1, OUT_DIM), brow3, bcol3)
    return (z, z_g)


# speedup vs baseline: 20.2594x; 1.0022x over previous
import functools

import jax
import jax.numpy as jnp
from jax import lax
from jax.experimental import pallas as pl
from jax.experimental.pallas import tpu as pltpu
from jax.experimental.pallas import tpu_sc as plsc

N = 10000
E = 320000
IN_DIM = 128
HIDDEN = 256
OUT_DIM = 128
B = 64

NSUB = 16
K = 128
NCH = 160
E_PAD = NSUB * NCH * K
EDGES_PER_SUB = NCH * K
DEG_CHUNK = 5120
DEG_NCH = EDGES_PER_SUB // DEG_CHUNK
SINK = N
N_ACC = N + 16
ROWS_PER_IO_SUB = 1000

_f32 = jnp.float32


def _vector_mesh():
    return plsc.VectorSubcoreMesh(core_axis_name="c", subcore_axis_name="s")


def _deg_body(dst_hbm, deg_hbm, ones_v, idx_v, stage_v, acc_sh, sem):
    c = lax.axis_index("c")
    s = lax.axis_index("s")
    ones_v[...] = jnp.ones((DEG_CHUNK,), _f32)

    @pl.when((c == 0) & (s < 10))
    def _():
        pltpu.sync_copy(ones_v.at[pl.ds(0, ROWS_PER_IO_SUB)],
                        acc_sh.at[pl.ds(s * ROWS_PER_IO_SUB, ROWS_PER_IO_SUB)])

    plsc.subcore_barrier()

    @pl.when(c == 0)
    def _():
        @pl.loop(0, DEG_NCH)
        def _(j):
            pltpu.sync_copy(dst_hbm.at[s].at[j], idx_v)
            pltpu.sync_copy(ones_v, acc_sh.at[idx_v], add=True)

    plsc.subcore_barrier()

    @pl.when((c == 0) & (s < 10))
    def _():
        rows = pl.ds(s * ROWS_PER_IO_SUB, ROWS_PER_IO_SUB)
        pltpu.sync_copy(acc_sh.at[rows], stage_v)
        pltpu.sync_copy(stage_v, deg_hbm.at[rows])


def _degrees(dst_d):
    kern = pl.kernel(
        _deg_body,
        out_type=jax.ShapeDtypeStruct((N,), _f32),
        mesh=_vector_mesh(),
        scratch_types=[
            pltpu.VMEM((DEG_CHUNK,), _f32),
            pltpu.VMEM((DEG_CHUNK,), jnp.int32),
            pltpu.VMEM((ROWS_PER_IO_SUB,), _f32),
            pltpu.VMEM_SHARED((N_ACC,), _f32),
            pltpu.SemaphoreType.DMA,
        ],
    )
    return kern(dst_d)


def _pipelined_gather_scatter(table, src_rows, dst_rows, nch, acc_sh,
                              srcv, dstv, datav, gsems, isems, ssems):
    def idx_copies(j, b):
        return (pltpu.make_async_copy(src_rows.at[j], srcv[b], isems[b]),
                pltpu.make_async_copy(dst_rows.at[j], dstv[b], isems[b]))

    def start_idx(j, b):
        for cp in idx_copies(j, b):
            cp.start()

    def wait_idx(j, b):
        for cp in idx_copies(j, b):
            cp.wait()

    def gather(b):
        return pltpu.make_async_copy(table.at[srcv[b]], datav[b], gsems[b])

    def wait_scatter(b):
        pltpu.make_async_copy(table.at[pl.ds(0, K)], datav[b], ssems[b]).wait()

    start_idx(0, 0)
    start_idx(1, 1)
    wait_idx(0, 0)
    gather(0).start()
    plsc.subcore_barrier()

    @pl.loop(0, nch, step=2)
    def _(j):
        for b in range(2):
            jj = j + b
            nb = 1 - b
            gather(b).wait()

            @pl.when(jj + 1 < nch)
            def _():
                wait_idx(jj + 1, nb)

                @pl.when(jj >= 1)
                def _():
                    wait_scatter(nb)
                gather(nb).start()

            pltpu.async_copy(datav[b], acc_sh.at[dstv[b]], ssems[b], add=True)

            @pl.when(jj + 2 < nch)
            def _():
                start_idx(jj + 2, b)

    wait_scatter(0)
    wait_scatter(1)
    plsc.subcore_barrier()


def _scatter_body(half, hs_hbm, src_hbm, dst_hbm, agg_hbm,
                  src_v0, src_v1, dst_v0, dst_v1, data_v0, data_v1,
                  acc_sh, sem0, sem1, semi0, semi1, sems0, sems1):
    c = lax.axis_index("c")
    s = lax.axis_index("s")
    table = hs_hbm.at[c]

    @pl.when(s < 10)
    def _():
        rows = pl.ds(s * ROWS_PER_IO_SUB, ROWS_PER_IO_SUB)
        pltpu.sync_copy(table.at[rows], acc_sh.at[rows])

    _pipelined_gather_scatter(table, src_hbm.at[s], dst_hbm.at[s], NCH, acc_sh,
                              (src_v0, src_v1), (dst_v0, dst_v1),
                              (data_v0, data_v1), (sem0, sem1), (semi0, semi1),
                              (sems0, sems1))

    @pl.when(s < 10)
    def _():
        rows = pl.ds(s * ROWS_PER_IO_SUB, ROWS_PER_IO_SUB)
        pltpu.sync_copy(acc_sh.at[rows], agg_hbm.at[c].at[rows])


def _edge_aggregate(hs, src_g, dst_g, half):
    kern = pl.kernel(
        functools.partial(_scatter_body, half),
        out_type=jax.ShapeDtypeStruct((2, N, half), _f32),
        mesh=_vector_mesh(),
        scratch_types=[
            pltpu.VMEM((K,), jnp.int32),
            pltpu.VMEM((K,), jnp.int32),
            pltpu.VMEM((K,), jnp.int32),
            pltpu.VMEM((K,), jnp.int32),
            pltpu.VMEM((K, half), _f32),
            pltpu.VMEM((K, half), _f32),
            pltpu.VMEM_SHARED((N_ACC, half), _f32),
            pltpu.SemaphoreType.DMA,
            pltpu.SemaphoreType.DMA,
            pltpu.SemaphoreType.DMA,
            pltpu.SemaphoreType.DMA,
            pltpu.SemaphoreType.DMA,
            pltpu.SemaphoreType.DMA,
        ],
    )
    return kern(hs, src_g, dst_g)


NCH2 = E_PAD // (2 * NSUB * K)


def _scatter2_body(hs_hbm, src_hbm, dst_hbm, agg_hbm,
                   src_v0, src_v1, dst_v0, dst_v1, data_v0, data_v1,
                   acc_sh, sem0, sem1, semi0, semi1, sems0, sems1):
    c = lax.axis_index("c")
    s = lax.axis_index("s")
    w = c * NSUB + s

    @pl.when(s < 10)
    def _():
        rows = pl.ds(s * ROWS_PER_IO_SUB, ROWS_PER_IO_SUB)

        @pl.when(c == 0)
        def _():
            pltpu.sync_copy(hs_hbm.at[rows], acc_sh.at[rows])

        @pl.when(c == 1)
        def _():
            data_v0[...] = jnp.zeros((K, OUT_DIM), _f32)
            @pl.loop(0, 8)
            def _(k):
                pltpu.sync_copy(
                    data_v0.at[pl.ds(0, 125)],
                    acc_sh.at[pl.ds(s * ROWS_PER_IO_SUB + k * 125, 125)])

    _pipelined_gather_scatter(hs_hbm, src_hbm.at[w], dst_hbm.at[w], NCH2,
                              acc_sh, (src_v0, src_v1), (dst_v0, dst_v1),
                              (data_v0, data_v1), (sem0, sem1), (semi0, semi1),
                              (sems0, sems1))

    @pl.when(s < 10)
    def _():
        rows = pl.ds(s * ROWS_PER_IO_SUB, ROWS_PER_IO_SUB)
        pltpu.sync_copy(acc_sh.at[rows], agg_hbm.at[c].at[rows])


def _edge_aggregate2(hs, src_g2, dst_g2):
    kern = pl.kernel(
        _scatter2_body,
        out_type=jax.ShapeDtypeStruct((2, N, OUT_DIM), _f32),
        mesh=_vector_mesh(),
        scratch_types=[
            pltpu.VMEM((K,), jnp.int32),
            pltpu.VMEM((K,), jnp.int32),
            pltpu.VMEM((K,), jnp.int32),
            pltpu.VMEM((K,), jnp.int32),
            pltpu.VMEM((K, OUT_DIM), _f32),
            pltpu.VMEM((K, OUT_DIM), _f32),
            pltpu.VMEM_SHARED((N_ACC, OUT_DIM), _f32),
            pltpu.SemaphoreType.DMA,
            pltpu.SemaphoreType.DMA,
            pltpu.SemaphoreType.DMA,
            pltpu.SemaphoreType.DMA,
            pltpu.SemaphoreType.DMA,
            pltpu.SemaphoreType.DMA,
        ],
    )
    return kern(hs, src_g2, dst_g2)


def _mm1_body(x_ref, deg_ref, w_ref, out_ref):
    sc = lax.rsqrt(deg_ref[...])
    out_ref[0] = jnp.dot(x_ref[...] * sc, w_ref[...],
                         preferred_element_type=_f32,
                         precision=lax.Precision.HIGHEST)


def _mm1(x, deg2d, W1):
    grid = (N // 1000, 2)
    return pl.pallas_call(
        _mm1_body,
        grid=grid,
        in_specs=[
            pl.BlockSpec((1000, IN_DIM), lambda i, c: (i, 0)),
            pl.BlockSpec((1000, 1), lambda i, c: (i, 0)),
            pl.BlockSpec((IN_DIM, HIDDEN // 2), lambda i, c: (0, c)),
        ],
        out_specs=pl.BlockSpec((1, 1000, HIDDEN // 2), lambda i, c: (c, i, 0)),
        out_shape=jax.ShapeDtypeStruct((2, N, HIDDEN // 2), _f32),
    )(x, deg2d, W1)


def _mm2_body(agg_ref, deg_ref, b1_ref, w_ref, out_ref):
    sc = lax.rsqrt(deg_ref[...])
    acat = jnp.concatenate([agg_ref[0], agg_ref[1]], axis=1)
    u = sc * jax.nn.relu(sc * acat + b1_ref[...])
    out_ref[...] = jnp.dot(u, w_ref[...],
                           preferred_element_type=_f32,
                           precision=lax.Precision.HIGHEST)


def _mm2(agg1, deg2d, b1, W2):
    grid = (N // 1000,)
    return pl.pallas_call(
        _mm2_body,
        grid=grid,
        in_specs=[
            pl.BlockSpec((2, 1000, HIDDEN // 2), lambda i: (0, i, 0)),
            pl.BlockSpec((1000, 1), lambda i: (i, 0)),
            pl.BlockSpec((1, HIDDEN), lambda i: (0, 0)),
            pl.BlockSpec((HIDDEN, OUT_DIM), lambda i: (0, 0)),
        ],
        out_specs=pl.BlockSpec((1000, OUT_DIM), lambda i: (i, 0)),
        out_shape=jax.ShapeDtypeStruct((N, OUT_DIM), _f32),
    )(agg1, deg2d, b1, W2)


def _pool_body(agg_ref, deg_ref, b2_ref, brow_ref, bcol_ref,
               z_ref, zg_ref, sums_scr, cnt_scr, mx_scr):
    i = pl.program_id(0)
    nblk = pl.num_programs(0)
    sc = lax.rsqrt(deg_ref[...])
    acat = agg_ref[0] + agg_ref[1]
    z = sc * acat + b2_ref[...]
    z_ref[...] = z

    @pl.when(i == 0)
    def _():
        sums_scr[...] = jnp.zeros_like(sums_scr)
        cnt_scr[...] = jnp.zeros_like(cnt_scr)
        mx_scr[...] = jnp.full_like(mx_scr, -jnp.inf)

    brow = brow_ref[0]
    seg_ids = lax.broadcasted_iota(jnp.int32, (B, 1), 0)
    onehot = (brow == seg_ids).astype(_f32)
    sums_scr[...] += jnp.dot(onehot, z, preferred_element_type=_f32,
                             precision=lax.Precision.HIGHEST)
    cnt_scr[...] += jnp.sum(onehot, axis=1, keepdims=True)

    bcol = bcol_ref[0]
    for b in range(B):
        masked = jnp.where(bcol == b, z, -jnp.inf)
        row = jnp.max(masked, axis=0, keepdims=True)
        mx_scr[pl.ds(b, 1), :] = jnp.maximum(mx_scr[pl.ds(b, 1), :], row)

    @pl.when(i == nblk - 1)
    def _():
        mean = sums_scr[...] / jnp.maximum(cnt_scr[...], 1.0)
        zg_ref[:, :OUT_DIM] = mean
        zg_ref[:, OUT_DIM:] = mx_scr[...]


def _pool(agg2, deg2d, b2, brow3, bcol3):
    grid = (N // 1000,)
    return pl.pallas_call(
        _pool_body,
        grid=grid,
        in_specs=[
            pl.BlockSpec((2, 1000, OUT_DIM), lambda i: (0, i, 0)),
            pl.BlockSpec((1000, 1), lambda i: (i, 0)),
            pl.BlockSpec((1, OUT_DIM), lambda i: (0, 0)),
            pl.BlockSpec((1, 1, 1000), lambda i: (i, 0, 0)),
            pl.BlockSpec((1, 1000, 1), lambda i: (i, 0, 0)),
        ],
        out_specs=[
            pl.BlockSpec((1000, OUT_DIM), lambda i: (i, 0)),
            pl.BlockSpec((B, 2 * OUT_DIM), lambda i: (0, 0)),
        ],
        out_shape=[
            jax.ShapeDtypeStruct((N, OUT_DIM), _f32),
            jax.ShapeDtypeStruct((B, 2 * OUT_DIM), _f32),
        ],
        scratch_shapes=[
            pltpu.VMEM((B, OUT_DIM), _f32),
            pltpu.VMEM((B, 1), _f32),
            pltpu.VMEM((B, OUT_DIM), _f32),
        ],
    )(agg2, deg2d, b2, brow3, bcol3)


def kernel(x, edge_index, batch, W1, b1, W2, b2):
    pad = E_PAD - E
    pad_src = jnp.arange(pad, dtype=jnp.int32) % N
    pad_dst = SINK + (jnp.arange(pad, dtype=jnp.int32) % (N_ACC - SINK))
    src = jnp.concatenate([edge_index[0], pad_src])
    dst = jnp.concatenate([edge_index[1], pad_dst])
    src_g = src.reshape(NSUB, NCH, K)
    dst_g = dst.reshape(NSUB, NCH, K)
    src_g2 = src.reshape(2 * NSUB, NCH2, K)
    dst_g2 = dst.reshape(2 * NSUB, NCH2, K)
    dst_d = dst.reshape(NSUB, DEG_NCH, DEG_CHUNK)

    deg = _degrees(dst_d)
    deg2d = deg.reshape(N, 1)

    hs1 = _mm1(x, deg2d, W1)
    agg1 = _edge_aggregate(hs1, src_g, dst_g, HIDDEN // 2)
    hs2 = _mm2(agg1, deg2d, b1.reshape(1, HIDDEN), W2)
    agg2 = _edge_aggregate2(hs2, src_g2, dst_g2)

    brow3 = batch.reshape(N // 1000, 1, 1000)
    bcol3 = batch.reshape(N // 1000, 1000, 1)
    z, z_g = _pool(agg2, deg2d, b2.reshape(1, OUT_DIM), brow3, bcol3)
    return (z, z_g)


# speedup vs baseline: 22.9903x; 1.1348x over previous
import functools

import jax
import jax.numpy as jnp
from jax import lax
from jax.experimental import pallas as pl
from jax.experimental.pallas import tpu as pltpu
from jax.experimental.pallas import tpu_sc as plsc

N = 10000
E = 320000
IN_DIM = 128
HIDDEN = 256
OUT_DIM = 128
B = 64

NSUB = 16
K = 128
NCH = 160
E_PAD = NSUB * NCH * K
EDGES_PER_SUB = NCH * K
DEG_CHUNK = 5120
DEG_NCH = EDGES_PER_SUB // DEG_CHUNK
SINK = N
N_ACC = N + 16
ROWS_PER_IO_SUB = 1000

_f32 = jnp.float32


def _vector_mesh():
    return plsc.VectorSubcoreMesh(core_axis_name="c", subcore_axis_name="s")


def _deg_body(dst_hbm, deg_hbm, ones_v, idx_v, stage_v, acc_sh, sem):
    c = lax.axis_index("c")
    s = lax.axis_index("s")
    ones_v[...] = jnp.ones((DEG_CHUNK,), _f32)

    @pl.when((c == 0) & (s < 10))
    def _():
        pltpu.sync_copy(ones_v.at[pl.ds(0, ROWS_PER_IO_SUB)],
                        acc_sh.at[pl.ds(s * ROWS_PER_IO_SUB, ROWS_PER_IO_SUB)])

    plsc.subcore_barrier()

    @pl.when(c == 0)
    def _():
        @pl.loop(0, DEG_NCH)
        def _(j):
            pltpu.sync_copy(dst_hbm.at[s].at[j], idx_v)
            pltpu.sync_copy(ones_v, acc_sh.at[idx_v], add=True)

    plsc.subcore_barrier()

    @pl.when((c == 0) & (s < 10))
    def _():
        rows = pl.ds(s * ROWS_PER_IO_SUB, ROWS_PER_IO_SUB)
        pltpu.sync_copy(acc_sh.at[rows], stage_v)
        pltpu.sync_copy(stage_v, deg_hbm.at[rows])


def _degrees(dst_d):
    kern = pl.kernel(
        _deg_body,
        out_type=jax.ShapeDtypeStruct((N,), _f32),
        mesh=_vector_mesh(),
        scratch_types=[
            pltpu.VMEM((DEG_CHUNK,), _f32),
            pltpu.VMEM((DEG_CHUNK,), jnp.int32),
            pltpu.VMEM((ROWS_PER_IO_SUB,), _f32),
            pltpu.VMEM_SHARED((N_ACC,), _f32),
            pltpu.SemaphoreType.DMA,
        ],
    )
    return kern(dst_d)


def _pipelined_gather_scatter(table, src_rows, dst_rows, nch, acc_sh,
                              srcv, dstv, datav, gsems, isems, ssems):
    def idx_copies(j, b):
        return (pltpu.make_async_copy(src_rows.at[j], srcv[b], isems[b]),
                pltpu.make_async_copy(dst_rows.at[j], dstv[b], isems[b]))

    def start_idx(j, b):
        for cp in idx_copies(j, b):
            cp.start()

    def wait_idx(j, b):
        for cp in idx_copies(j, b):
            cp.wait()

    def gather(b):
        return pltpu.make_async_copy(table.at[srcv[b]], datav[b], gsems[b])

    def wait_scatter(b):
        pltpu.make_async_copy(table.at[pl.ds(0, K)], datav[b], ssems[b]).wait()

    start_idx(0, 0)
    start_idx(1, 1)
    wait_idx(0, 0)
    gather(0).start()
    plsc.subcore_barrier()

    @pl.loop(0, nch, step=2)
    def _(j):
        for b in range(2):
            jj = j + b
            nb = 1 - b
            gather(b).wait()

            @pl.when(jj + 1 < nch)
            def _():
                wait_idx(jj + 1, nb)

                @pl.when(jj >= 1)
                def _():
                    wait_scatter(nb)
                gather(nb).start()

            pltpu.async_copy(datav[b], acc_sh.at[dstv[b]], ssems[b], add=True)

            @pl.when(jj + 2 < nch)
            def _():
                start_idx(jj + 2, b)

    wait_scatter(0)
    wait_scatter(1)
    plsc.subcore_barrier()


def _scatter_body(half, hs_hbm, src_hbm, dst_hbm, agg_hbm,
                  src_v0, src_v1, dst_v0, dst_v1, data_v0, data_v1,
                  acc_sh, sem0, sem1, semi0, semi1, sems0, sems1):
    c = lax.axis_index("c")
    s = lax.axis_index("s")
    table = hs_hbm.at[c]

    @pl.when(s < 10)
    def _():
        rows = pl.ds(s * ROWS_PER_IO_SUB, ROWS_PER_IO_SUB)
        pltpu.sync_copy(table.at[rows], acc_sh.at[rows])

    _pipelined_gather_scatter(table, src_hbm.at[s], dst_hbm.at[s], NCH, acc_sh,
                              (src_v0, src_v1), (dst_v0, dst_v1),
                              (data_v0, data_v1), (sem0, sem1), (semi0, semi1),
                              (sems0, sems1))

    @pl.when(s < 10)
    def _():
        rows = pl.ds(s * ROWS_PER_IO_SUB, ROWS_PER_IO_SUB)
        pltpu.sync_copy(acc_sh.at[rows], agg_hbm.at[c].at[rows])


def _edge_aggregate(hs, src_g, dst_g, half):
    kern = pl.kernel(
        functools.partial(_scatter_body, half),
        out_type=jax.ShapeDtypeStruct((2, N, half), _f32),
        mesh=_vector_mesh(),
        scratch_types=[
            pltpu.VMEM((K,), jnp.int32),
            pltpu.VMEM((K,), jnp.int32),
            pltpu.VMEM((K,), jnp.int32),
            pltpu.VMEM((K,), jnp.int32),
            pltpu.VMEM((K, half), _f32),
            pltpu.VMEM((K, half), _f32),
            pltpu.VMEM_SHARED((N_ACC, half), _f32),
            pltpu.SemaphoreType.DMA,
            pltpu.SemaphoreType.DMA,
            pltpu.SemaphoreType.DMA,
            pltpu.SemaphoreType.DMA,
            pltpu.SemaphoreType.DMA,
            pltpu.SemaphoreType.DMA,
        ],
    )
    return kern(hs, src_g, dst_g)


NCH2 = E_PAD // (2 * NSUB * K)


def _scatter2_body(hs_hbm, src_hbm, dst_hbm, agg_hbm,
                   src_v0, src_v1, dst_v0, dst_v1, data_v0, data_v1,
                   acc_sh, sem0, sem1, semi0, semi1, sems0, sems1):
    c = lax.axis_index("c")
    s = lax.axis_index("s")
    w = c * NSUB + s

    @pl.when(s < 10)
    def _():
        rows = pl.ds(s * ROWS_PER_IO_SUB, ROWS_PER_IO_SUB)

        @pl.when(c == 0)
        def _():
            pltpu.sync_copy(hs_hbm.at[rows], acc_sh.at[rows])

        @pl.when(c == 1)
        def _():
            data_v0[...] = jnp.zeros((K, OUT_DIM), _f32)
            @pl.loop(0, 8)
            def _(k):
                pltpu.sync_copy(
                    data_v0.at[pl.ds(0, 125)],
                    acc_sh.at[pl.ds(s * ROWS_PER_IO_SUB + k * 125, 125)])

    _pipelined_gather_scatter(hs_hbm, src_hbm.at[w], dst_hbm.at[w], NCH2,
                              acc_sh, (src_v0, src_v1), (dst_v0, dst_v1),
                              (data_v0, data_v1), (sem0, sem1), (semi0, semi1),
                              (sems0, sems1))

    @pl.when(s < 10)
    def _():
        rows = pl.ds(s * ROWS_PER_IO_SUB, ROWS_PER_IO_SUB)
        pltpu.sync_copy(acc_sh.at[rows], agg_hbm.at[c].at[rows])


def _edge_aggregate2(hs, src_g2, dst_g2):
    kern = pl.kernel(
        _scatter2_body,
        out_type=jax.ShapeDtypeStruct((2, N, OUT_DIM), _f32),
        mesh=_vector_mesh(),
        scratch_types=[
            pltpu.VMEM((K,), jnp.int32),
            pltpu.VMEM((K,), jnp.int32),
            pltpu.VMEM((K,), jnp.int32),
            pltpu.VMEM((K,), jnp.int32),
            pltpu.VMEM((K, OUT_DIM), _f32),
            pltpu.VMEM((K, OUT_DIM), _f32),
            pltpu.VMEM_SHARED((N_ACC, OUT_DIM), _f32),
            pltpu.SemaphoreType.DMA,
            pltpu.SemaphoreType.DMA,
            pltpu.SemaphoreType.DMA,
            pltpu.SemaphoreType.DMA,
            pltpu.SemaphoreType.DMA,
            pltpu.SemaphoreType.DMA,
        ],
    )
    return kern(hs, src_g2, dst_g2)


def _mm1_body(x_ref, deg_ref, w_ref, out_ref):
    sc = lax.rsqrt(deg_ref[...])
    out_ref[0] = jnp.dot(x_ref[...] * sc, w_ref[...],
                         preferred_element_type=_f32,
                         precision=lax.Precision.HIGHEST)


def _mm1(x, deg2d, W1):
    grid = (N // 1000, 2)
    return pl.pallas_call(
        _mm1_body,
        grid=grid,
        in_specs=[
            pl.BlockSpec((1000, IN_DIM), lambda i, c: (i, 0)),
            pl.BlockSpec((1000, 1), lambda i, c: (i, 0)),
            pl.BlockSpec((IN_DIM, HIDDEN // 2), lambda i, c: (0, c)),
        ],
        out_specs=pl.BlockSpec((1, 1000, HIDDEN // 2), lambda i, c: (c, i, 0)),
        out_shape=jax.ShapeDtypeStruct((2, N, HIDDEN // 2), _f32),
    )(x, deg2d, W1)


def _mm2_body(agg_ref, deg_ref, b1_ref, w_ref, out_ref):
    sc = lax.rsqrt(deg_ref[...])
    acat = jnp.concatenate([agg_ref[0], agg_ref[1]], axis=1)
    u = sc * jax.nn.relu(sc * acat + b1_ref[...])
    out_ref[...] = jnp.dot(u, w_ref[...],
                           preferred_element_type=_f32,
                           precision=lax.Precision.HIGHEST)


def _mm2(agg1, deg2d, b1, W2):
    grid = (N // 1000,)
    return pl.pallas_call(
        _mm2_body,
        grid=grid,
        in_specs=[
            pl.BlockSpec((2, 1000, HIDDEN // 2), lambda i: (0, i, 0)),
            pl.BlockSpec((1000, 1), lambda i: (i, 0)),
            pl.BlockSpec((1, HIDDEN), lambda i: (0, 0)),
            pl.BlockSpec((HIDDEN, OUT_DIM), lambda i: (0, 0)),
        ],
        out_specs=pl.BlockSpec((1000, OUT_DIM), lambda i: (i, 0)),
        out_shape=jax.ShapeDtypeStruct((N, OUT_DIM), _f32),
    )(agg1, deg2d, b1, W2)


def _pool_body(agg_ref, deg_ref, b2_ref, brow_ref, bcol_ref,
               z_ref, zg_ref, sums_scr, cnt_scr, mx_scr):
    i = pl.program_id(0)
    nblk = pl.num_programs(0)
    sc = lax.rsqrt(deg_ref[...])
    acat = agg_ref[0] + agg_ref[1]
    z = sc * acat + b2_ref[...]
    z_ref[...] = z

    @pl.when(i == 0)
    def _():
        sums_scr[...] = jnp.zeros_like(sums_scr)
        cnt_scr[...] = jnp.zeros_like(cnt_scr)
        mx_scr[...] = jnp.full_like(mx_scr, -jnp.inf)

    brow = brow_ref[0]
    seg_ids = lax.broadcasted_iota(jnp.int32, (B, 1), 0)
    onehot = (brow == seg_ids).astype(_f32)
    sums_scr[...] += jnp.dot(onehot, z, preferred_element_type=_f32,
                             precision=lax.Precision.HIGHEST)
    cnt_scr[...] += jnp.sum(onehot, axis=1, keepdims=True)

    bcol = bcol_ref[0]
    b_lo = brow_ref[0, 0, 0]
    b_hi = brow_ref[0, 0, 999]

    def _seg_max(b, _):
        masked = jnp.where(bcol == b, z, -jnp.inf)
        row = jnp.max(masked, axis=0, keepdims=True)
        mx_scr[pl.ds(b, 1), :] = jnp.maximum(mx_scr[pl.ds(b, 1), :], row)
        return _

    lax.fori_loop(b_lo, b_hi + 1, _seg_max, None)

    @pl.when(i == nblk - 1)
    def _():
        mean = sums_scr[...] / jnp.maximum(cnt_scr[...], 1.0)
        zg_ref[:, :OUT_DIM] = mean
        zg_ref[:, OUT_DIM:] = mx_scr[...]


def _pool(agg2, deg2d, b2, brow3, bcol3):
    grid = (N // 1000,)
    return pl.pallas_call(
        _pool_body,
        grid=grid,
        in_specs=[
            pl.BlockSpec((2, 1000, OUT_DIM), lambda i: (0, i, 0)),
            pl.BlockSpec((1000, 1), lambda i: (i, 0)),
            pl.BlockSpec((1, OUT_DIM), lambda i: (0, 0)),
            pl.BlockSpec((1, 1, 1000), lambda i: (i, 0, 0)),
            pl.BlockSpec((1, 1000, 1), lambda i: (i, 0, 0)),
        ],
        out_specs=[
            pl.BlockSpec((1000, OUT_DIM), lambda i: (i, 0)),
            pl.BlockSpec((B, 2 * OUT_DIM), lambda i: (0, 0)),
        ],
        out_shape=[
            jax.ShapeDtypeStruct((N, OUT_DIM), _f32),
            jax.ShapeDtypeStruct((B, 2 * OUT_DIM), _f32),
        ],
        scratch_shapes=[
            pltpu.VMEM((B, OUT_DIM), _f32),
            pltpu.VMEM((B, 1), _f32),
            pltpu.VMEM((B, OUT_DIM), _f32),
        ],
    )(agg2, deg2d, b2, brow3, bcol3)


def kernel(x, edge_index, batch, W1, b1, W2, b2):
    pad = E_PAD - E
    pad_src = jnp.arange(pad, dtype=jnp.int32) % N
    pad_dst = SINK + (jnp.arange(pad, dtype=jnp.int32) % (N_ACC - SINK))
    src = jnp.concatenate([edge_index[0], pad_src])
    dst = jnp.concatenate([edge_index[1], pad_dst])
    src_g = src.reshape(NSUB, NCH, K)
    dst_g = dst.reshape(NSUB, NCH, K)
    src_g2 = src.reshape(2 * NSUB, NCH2, K)
    dst_g2 = dst.reshape(2 * NSUB, NCH2, K)
    dst_d = dst.reshape(NSUB, DEG_NCH, DEG_CHUNK)

    deg = _degrees(dst_d)
    deg2d = deg.reshape(N, 1)

    hs1 = _mm1(x, deg2d, W1)
    agg1 = _edge_aggregate(hs1, src_g, dst_g, HIDDEN // 2)
    hs2 = _mm2(agg1, deg2d, b1.reshape(1, HIDDEN), W2)
    agg2 = _edge_aggregate2(hs2, src_g2, dst_g2)

    brow3 = batch.reshape(N // 1000, 1, 1000)
    bcol3 = batch.reshape(N // 1000, 1000, 1)
    z, z_g = _pool(agg2, deg2d, b2.reshape(1, OUT_DIM), brow3, bcol3)
    return (z, z_g)


# speedup vs baseline: 23.0155x; 1.0011x over previous
import functools

import jax
import jax.numpy as jnp
from jax import lax
from jax.experimental import pallas as pl
from jax.experimental.pallas import tpu as pltpu
from jax.experimental.pallas import tpu_sc as plsc

N = 10000
E = 320000
IN_DIM = 128
HIDDEN = 256
OUT_DIM = 128
B = 64

NSUB = 16
K = 128
NCH = 160
E_PAD = NSUB * NCH * K
EDGES_PER_SUB = NCH * K
DEG_CHUNK = 5120
DEG_NCH = EDGES_PER_SUB // DEG_CHUNK
SINK = N
N_ACC = N + 16
ROWS_PER_IO_SUB = 1000

_f32 = jnp.float32


def _vector_mesh():
    return plsc.VectorSubcoreMesh(core_axis_name="c", subcore_axis_name="s")


def _deg_body(dst_hbm, deg_hbm, ones_v, idx_v, stage_v, acc_sh, sem):
    c = lax.axis_index("c")
    s = lax.axis_index("s")
    ones_v[...] = jnp.ones((DEG_CHUNK,), _f32)

    @pl.when((c == 0) & (s < 10))
    def _():
        pltpu.sync_copy(ones_v.at[pl.ds(0, ROWS_PER_IO_SUB)],
                        acc_sh.at[pl.ds(s * ROWS_PER_IO_SUB, ROWS_PER_IO_SUB)])

    plsc.subcore_barrier()

    @pl.when(c == 0)
    def _():
        @pl.loop(0, DEG_NCH)
        def _(j):
            pltpu.sync_copy(dst_hbm.at[s].at[j], idx_v)
            pltpu.sync_copy(ones_v, acc_sh.at[idx_v], add=True)

    plsc.subcore_barrier()

    @pl.when((c == 0) & (s < 10))
    def _():
        rows = pl.ds(s * ROWS_PER_IO_SUB, ROWS_PER_IO_SUB)
        pltpu.sync_copy(acc_sh.at[rows], stage_v)
        pltpu.sync_copy(stage_v, deg_hbm.at[rows])


def _degrees(dst_d):
    kern = pl.kernel(
        _deg_body,
        out_type=jax.ShapeDtypeStruct((N,), _f32),
        mesh=_vector_mesh(),
        scratch_types=[
            pltpu.VMEM((DEG_CHUNK,), _f32),
            pltpu.VMEM((DEG_CHUNK,), jnp.int32),
            pltpu.VMEM((ROWS_PER_IO_SUB,), _f32),
            pltpu.VMEM_SHARED((N_ACC,), _f32),
            pltpu.SemaphoreType.DMA,
        ],
    )
    return kern(dst_d)


def _pipelined_gather_scatter(table, src_rows, dst_rows, nch, acc_sh,
                              srcv, dstv, datav, gsems, isems, ssems):
    def idx_copies(j, b):
        return (pltpu.make_async_copy(src_rows.at[j], srcv[b], isems[b]),
                pltpu.make_async_copy(dst_rows.at[j], dstv[b], isems[b]))

    def start_idx(j, b):
        for cp in idx_copies(j, b):
            cp.start()

    def wait_idx(j, b):
        for cp in idx_copies(j, b):
            cp.wait()

    def gather(b):
        return pltpu.make_async_copy(table.at[srcv[b]], datav[b], gsems[b])

    def wait_scatter(b):
        pltpu.make_async_copy(table.at[pl.ds(0, K)], datav[b], ssems[b]).wait()

    start_idx(0, 0)
    start_idx(1, 1)
    wait_idx(0, 0)
    gather(0).start()
    plsc.subcore_barrier()

    @pl.loop(0, nch, step=2)
    def _(j):
        for b in range(2):
            jj = j + b
            nb = 1 - b
            gather(b).wait()

            @pl.when(jj + 1 < nch)
            def _():
                wait_idx(jj + 1, nb)

                @pl.when(jj >= 1)
                def _():
                    wait_scatter(nb)
                gather(nb).start()

            pltpu.async_copy(datav[b], acc_sh.at[dstv[b]], ssems[b], add=True)

            @pl.when(jj + 2 < nch)
            def _():
                start_idx(jj + 2, b)

    wait_scatter(0)
    wait_scatter(1)
    plsc.subcore_barrier()


def _scatter_body(half, hs_hbm, src_hbm, dst_hbm, agg_hbm,
                  src_v0, src_v1, dst_v0, dst_v1, data_v0, data_v1,
                  acc_sh, sem0, sem1, semi0, semi1, sems0, sems1):
    c = lax.axis_index("c")
    s = lax.axis_index("s")
    table = hs_hbm.at[c]

    @pl.when(s < 10)
    def _():
        rows = pl.ds(s * ROWS_PER_IO_SUB, ROWS_PER_IO_SUB)
        pltpu.sync_copy(table.at[rows], acc_sh.at[rows])

    _pipelined_gather_scatter(table, src_hbm.at[s], dst_hbm.at[s], NCH, acc_sh,
                              (src_v0, src_v1), (dst_v0, dst_v1),
                              (data_v0, data_v1), (sem0, sem1), (semi0, semi1),
                              (sems0, sems1))

    @pl.when(s < 10)
    def _():
        rows = pl.ds(s * ROWS_PER_IO_SUB, ROWS_PER_IO_SUB)
        pltpu.sync_copy(acc_sh.at[rows], agg_hbm.at[c].at[rows])


def _edge_aggregate(hs, src_g, dst_g, half):
    kern = pl.kernel(
        functools.partial(_scatter_body, half),
        out_type=jax.ShapeDtypeStruct((2, N, half), _f32),
        mesh=_vector_mesh(),
        scratch_types=[
            pltpu.VMEM((K,), jnp.int32),
            pltpu.VMEM((K,), jnp.int32),
            pltpu.VMEM((K,), jnp.int32),
            pltpu.VMEM((K,), jnp.int32),
            pltpu.VMEM((K, half), _f32),
            pltpu.VMEM((K, half), _f32),
            pltpu.VMEM_SHARED((N_ACC, half), _f32),
            pltpu.SemaphoreType.DMA,
            pltpu.SemaphoreType.DMA,
            pltpu.SemaphoreType.DMA,
            pltpu.SemaphoreType.DMA,
            pltpu.SemaphoreType.DMA,
            pltpu.SemaphoreType.DMA,
        ],
    )
    return kern(hs, src_g, dst_g)


NCH2 = E_PAD // (2 * NSUB * K)


def _scatter2_body(hs_hbm, src_hbm, dst_hbm, agg_hbm,
                   src_v0, src_v1, dst_v0, dst_v1, data_v0, data_v1,
                   acc_sh, sem0, sem1, semi0, semi1, sems0, sems1):
    c = lax.axis_index("c")
    s = lax.axis_index("s")
    w = c * NSUB + s

    @pl.when(s < 10)
    def _():
        rows = pl.ds(s * ROWS_PER_IO_SUB, ROWS_PER_IO_SUB)

        @pl.when(c == 0)
        def _():
            pltpu.sync_copy(hs_hbm.at[rows], acc_sh.at[rows])

        @pl.when(c == 1)
        def _():
            data_v0[...] = jnp.zeros((K, OUT_DIM), _f32)
            @pl.loop(0, 8)
            def _(k):
                pltpu.sync_copy(
                    data_v0.at[pl.ds(0, 125)],
                    acc_sh.at[pl.ds(s * ROWS_PER_IO_SUB + k * 125, 125)])

    _pipelined_gather_scatter(hs_hbm, src_hbm.at[w], dst_hbm.at[w], NCH2,
                              acc_sh, (src_v0, src_v1), (dst_v0, dst_v1),
                              (data_v0, data_v1), (sem0, sem1), (semi0, semi1),
                              (sems0, sems1))

    @pl.when(s < 10)
    def _():
        rows = pl.ds(s * ROWS_PER_IO_SUB, ROWS_PER_IO_SUB)
        pltpu.sync_copy(acc_sh.at[rows], agg_hbm.at[c].at[rows])


def _edge_aggregate2(hs, src_g2, dst_g2):
    kern = pl.kernel(
        _scatter2_body,
        out_type=jax.ShapeDtypeStruct((2, N, OUT_DIM), _f32),
        mesh=_vector_mesh(),
        scratch_types=[
            pltpu.VMEM((K,), jnp.int32),
            pltpu.VMEM((K,), jnp.int32),
            pltpu.VMEM((K,), jnp.int32),
            pltpu.VMEM((K,), jnp.int32),
            pltpu.VMEM((K, OUT_DIM), _f32),
            pltpu.VMEM((K, OUT_DIM), _f32),
            pltpu.VMEM_SHARED((N_ACC, OUT_DIM), _f32),
            pltpu.SemaphoreType.DMA,
            pltpu.SemaphoreType.DMA,
            pltpu.SemaphoreType.DMA,
            pltpu.SemaphoreType.DMA,
            pltpu.SemaphoreType.DMA,
            pltpu.SemaphoreType.DMA,
        ],
    )
    return kern(hs, src_g2, dst_g2)


def _mm1_body(x_ref, deg_ref, w_ref, out_ref):
    sc = lax.rsqrt(deg_ref[...])
    out_ref[0] = jnp.dot(x_ref[...] * sc, w_ref[...],
                         preferred_element_type=_f32,
                         precision=lax.Precision.HIGHEST)


def _mm1(x, deg2d, W1):
    grid = (N // 1000, 2)
    return pl.pallas_call(
        _mm1_body,
        grid=grid,
        in_specs=[
            pl.BlockSpec((1000, IN_DIM), lambda i, c: (i, 0)),
            pl.BlockSpec((1000, 1), lambda i, c: (i, 0)),
            pl.BlockSpec((IN_DIM, HIDDEN // 2), lambda i, c: (0, c)),
        ],
        out_specs=pl.BlockSpec((1, 1000, HIDDEN // 2), lambda i, c: (c, i, 0)),
        out_shape=jax.ShapeDtypeStruct((2, N, HIDDEN // 2), _f32),
        compiler_params=pltpu.CompilerParams(
            dimension_semantics=("parallel", "parallel")),
    )(x, deg2d, W1)


def _mm2_body(agg_ref, deg_ref, b1_ref, w_ref, out_ref):
    sc = lax.rsqrt(deg_ref[...])
    acat = jnp.concatenate([agg_ref[0], agg_ref[1]], axis=1)
    u = sc * jax.nn.relu(sc * acat + b1_ref[...])
    out_ref[...] = jnp.dot(u, w_ref[...],
                           preferred_element_type=_f32,
                           precision=lax.Precision.HIGHEST)


def _mm2(agg1, deg2d, b1, W2):
    grid = (N // 1000,)
    return pl.pallas_call(
        _mm2_body,
        grid=grid,
        in_specs=[
            pl.BlockSpec((2, 1000, HIDDEN // 2), lambda i: (0, i, 0)),
            pl.BlockSpec((1000, 1), lambda i: (i, 0)),
            pl.BlockSpec((1, HIDDEN), lambda i: (0, 0)),
            pl.BlockSpec((HIDDEN, OUT_DIM), lambda i: (0, 0)),
        ],
        out_specs=pl.BlockSpec((1000, OUT_DIM), lambda i: (i, 0)),
        out_shape=jax.ShapeDtypeStruct((N, OUT_DIM), _f32),
        compiler_params=pltpu.CompilerParams(
            dimension_semantics=("parallel",)),
    )(agg1, deg2d, b1, W2)


def _pool_body(agg_ref, deg_ref, b2_ref, brow_ref, bcol_ref,
               z_ref, zg_ref, sums_scr, cnt_scr, mx_scr):
    i = pl.program_id(0)
    nblk = pl.num_programs(0)
    sc = lax.rsqrt(deg_ref[...])
    acat = agg_ref[0] + agg_ref[1]
    z = sc * acat + b2_ref[...]
    z_ref[...] = z

    @pl.when(i == 0)
    def _():
        sums_scr[...] = jnp.zeros_like(sums_scr)
        cnt_scr[...] = jnp.zeros_like(cnt_scr)
        mx_scr[...] = jnp.full_like(mx_scr, -jnp.inf)

    brow = brow_ref[0]
    seg_ids = lax.broadcasted_iota(jnp.int32, (B, 1), 0)
    onehot = (brow == seg_ids).astype(_f32)
    sums_scr[...] += jnp.dot(onehot, z, preferred_element_type=_f32,
                             precision=lax.Precision.HIGHEST)
    cnt_scr[...] += jnp.sum(onehot, axis=1, keepdims=True)

    bcol = bcol_ref[0]
    b_lo = brow_ref[0, 0, 0]
    b_hi = brow_ref[0, 0, 999]

    def _seg_max(b, _):
        masked = jnp.where(bcol == b, z, -jnp.inf)
        row = jnp.max(masked, axis=0, keepdims=True)
        mx_scr[pl.ds(b, 1), :] = jnp.maximum(mx_scr[pl.ds(b, 1), :], row)
        return _

    lax.fori_loop(b_lo, b_hi + 1, _seg_max, None)

    @pl.when(i == nblk - 1)
    def _():
        mean = sums_scr[...] / jnp.maximum(cnt_scr[...], 1.0)
        zg_ref[:, :OUT_DIM] = mean
        zg_ref[:, OUT_DIM:] = mx_scr[...]


def _pool(agg2, deg2d, b2, brow3, bcol3):
    grid = (N // 1000,)
    return pl.pallas_call(
        _pool_body,
        grid=grid,
        in_specs=[
            pl.BlockSpec((2, 1000, OUT_DIM), lambda i: (0, i, 0)),
            pl.BlockSpec((1000, 1), lambda i: (i, 0)),
            pl.BlockSpec((1, OUT_DIM), lambda i: (0, 0)),
            pl.BlockSpec((1, 1, 1000), lambda i: (i, 0, 0)),
            pl.BlockSpec((1, 1000, 1), lambda i: (i, 0, 0)),
        ],
        out_specs=[
            pl.BlockSpec((1000, OUT_DIM), lambda i: (i, 0)),
            pl.BlockSpec((B, 2 * OUT_DIM), lambda i: (0, 0)),
        ],
        out_shape=[
            jax.ShapeDtypeStruct((N, OUT_DIM), _f32),
            jax.ShapeDtypeStruct((B, 2 * OUT_DIM), _f32),
        ],
        scratch_shapes=[
            pltpu.VMEM((B, OUT_DIM), _f32),
            pltpu.VMEM((B, 1), _f32),
            pltpu.VMEM((B, OUT_DIM), _f32),
        ],
    )(agg2, deg2d, b2, brow3, bcol3)


def kernel(x, edge_index, batch, W1, b1, W2, b2):
    pad = E_PAD - E
    pad_src = jnp.arange(pad, dtype=jnp.int32) % N
    pad_dst = SINK + (jnp.arange(pad, dtype=jnp.int32) % (N_ACC - SINK))
    src = jnp.concatenate([edge_index[0], pad_src])
    dst = jnp.concatenate([edge_index[1], pad_dst])
    src_g = src.reshape(NSUB, NCH, K)
    dst_g = dst.reshape(NSUB, NCH, K)
    src_g2 = src.reshape(2 * NSUB, NCH2, K)
    dst_g2 = dst.reshape(2 * NSUB, NCH2, K)
    dst_d = dst.reshape(NSUB, DEG_NCH, DEG_CHUNK)

    deg = _degrees(dst_d)
    deg2d = deg.reshape(N, 1)

    hs1 = _mm1(x, deg2d, W1)
    agg1 = _edge_aggregate(hs1, src_g, dst_g, HIDDEN // 2)
    hs2 = _mm2(agg1, deg2d, b1.reshape(1, HIDDEN), W2)
    agg2 = _edge_aggregate2(hs2, src_g2, dst_g2)

    brow3 = batch.reshape(N // 1000, 1, 1000)
    bcol3 = batch.reshape(N // 1000, 1000, 1)
    z, z_g = _pool(agg2, deg2d, b2.reshape(1, OUT_DIM), brow3, bcol3)
    return (z, z_g)


# speedup vs baseline: 27.8811x; 1.2114x over previous
import functools

import jax
import jax.numpy as jnp
from jax import lax
from jax.experimental import pallas as pl
from jax.experimental.pallas import tpu as pltpu
from jax.experimental.pallas import tpu_sc as plsc

N = 10000
E = 320000
IN_DIM = 128
HIDDEN = 256
OUT_DIM = 128
B = 64

NSUB = 16
K = 128
NCH = 168
E_PAD = NSUB * NCH * K
EDGES_PER_SUB = NCH * K
DEG_CHUNK = 5376
DEG_NCH = EDGES_PER_SUB // DEG_CHUNK
SINK = N
N_ACC = N + 16
ROWS_PER_IO_SUB = 1000

_f32 = jnp.float32


def _vector_mesh():
    return plsc.VectorSubcoreMesh(core_axis_name="c", subcore_axis_name="s")


def _deg_body(dst_hbm, deg_hbm, ones_v, idx_v, stage_v, acc_sh, sem):
    c = lax.axis_index("c")
    s = lax.axis_index("s")
    ones_v[...] = jnp.ones((DEG_CHUNK,), _f32)

    @pl.when((c == 0) & (s < 10))
    def _():
        pltpu.sync_copy(ones_v.at[pl.ds(0, ROWS_PER_IO_SUB)],
                        acc_sh.at[pl.ds(s * ROWS_PER_IO_SUB, ROWS_PER_IO_SUB)])

    plsc.subcore_barrier()

    @pl.when(c == 0)
    def _():
        @pl.loop(0, DEG_NCH)
        def _(j):
            pltpu.sync_copy(dst_hbm.at[s].at[j], idx_v)
            pltpu.sync_copy(ones_v, acc_sh.at[idx_v], add=True)

    plsc.subcore_barrier()

    @pl.when((c == 0) & (s < 10))
    def _():
        rows = pl.ds(s * ROWS_PER_IO_SUB, ROWS_PER_IO_SUB)
        pltpu.sync_copy(acc_sh.at[rows], stage_v)
        pltpu.sync_copy(stage_v, deg_hbm.at[rows])


def _degrees(dst_d):
    kern = pl.kernel(
        _deg_body,
        out_type=jax.ShapeDtypeStruct((N,), _f32),
        mesh=_vector_mesh(),
        scratch_types=[
            pltpu.VMEM((DEG_CHUNK,), _f32),
            pltpu.VMEM((DEG_CHUNK,), jnp.int32),
            pltpu.VMEM((ROWS_PER_IO_SUB,), _f32),
            pltpu.VMEM_SHARED((N_ACC,), _f32),
            pltpu.SemaphoreType.DMA,
        ],
    )
    return kern(dst_d)


def _pipelined_gather_scatter(table, src_rows, dst_rows, nch, acc_sh,
                              srcv, dstv, datav, gsems, isems, ssems):
    ND = 3
    NI = 4

    def idx_copies(j, bi):
        return (pltpu.make_async_copy(src_rows.at[j], srcv[bi], isems[bi]),
                pltpu.make_async_copy(dst_rows.at[j], dstv[bi], isems[bi]))

    def start_idx(j, bi):
        for cp in idx_copies(j, bi):
            cp.start()

    def wait_idx(j, bi):
        for cp in idx_copies(j, bi):
            cp.wait()

    def gather(bd, bi):
        return pltpu.make_async_copy(table.at[srcv[bi]], datav[bd], gsems[bd])

    def wait_scatter(bd):
        pltpu.make_async_copy(table.at[pl.ds(0, K)], datav[bd], ssems[bd]).wait()

    start_idx(0, 0)
    start_idx(1, 1)
    start_idx(2, 2)
    wait_idx(0, 0)
    gather(0, 0).start()
    wait_idx(1, 1)
    gather(1, 1).start()
    plsc.subcore_barrier()

    @pl.loop(0, nch, step=12)
    def _(j):
        for u in range(12):
            jj = j + u
            bd = u % ND
            bi = u % NI
            b2d = (u + 2) % ND
            b2i = (u + 2) % NI
            gather(bd, bi).wait()

            @pl.when(jj + 2 < nch)
            def _():
                wait_idx(jj + 2, b2i)

                @pl.when(jj >= 1)
                def _():
                    wait_scatter(b2d)
                gather(b2d, b2i).start()

                @pl.when(jj + 3 < nch)
                def _():
                    start_idx(jj + 3, (u + 3) % NI)

            pltpu.async_copy(datav[bd], acc_sh.at[dstv[bi]], ssems[bd],
                             add=True)

    wait_scatter(0)
    wait_scatter(1)
    wait_scatter(2)
    plsc.subcore_barrier()


def _scatter_body(half, hs_hbm, src_hbm, dst_hbm, agg_hbm,
                  srcv, dstv, datav, acc_sh, gsems, isems, ssems):
    c = lax.axis_index("c")
    s = lax.axis_index("s")
    table = hs_hbm.at[c]

    @pl.when(s < 10)
    def _():
        rows = pl.ds(s * ROWS_PER_IO_SUB, ROWS_PER_IO_SUB)
        pltpu.sync_copy(table.at[rows], acc_sh.at[rows])

    _pipelined_gather_scatter(table, src_hbm.at[s], dst_hbm.at[s], NCH, acc_sh,
                              srcv, dstv, datav, gsems, isems, ssems)

    @pl.when(s < 10)
    def _():
        rows = pl.ds(s * ROWS_PER_IO_SUB, ROWS_PER_IO_SUB)
        pltpu.sync_copy(acc_sh.at[rows], agg_hbm.at[c].at[rows])


def _edge_aggregate(hs, src_g, dst_g, half):
    kern = pl.kernel(
        functools.partial(_scatter_body, half),
        out_type=jax.ShapeDtypeStruct((2, N, half), _f32),
        mesh=_vector_mesh(),
        scratch_types=[
            tuple(pltpu.VMEM((K,), jnp.int32) for _ in range(4)),
            tuple(pltpu.VMEM((K,), jnp.int32) for _ in range(4)),
            tuple(pltpu.VMEM((K, half), _f32) for _ in range(3)),
            pltpu.VMEM_SHARED((N_ACC, half), _f32),
            tuple(pltpu.SemaphoreType.DMA for _ in range(3)),
            tuple(pltpu.SemaphoreType.DMA for _ in range(4)),
            tuple(pltpu.SemaphoreType.DMA for _ in range(3)),
        ],
    )
    return kern(hs, src_g, dst_g)


NCH2 = E_PAD // (2 * NSUB * K)


def _scatter2_body(hs_hbm, src_hbm, dst_hbm, agg_hbm,
                   srcv, dstv, datav, acc_sh, gsems, isems, ssems):
    c = lax.axis_index("c")
    s = lax.axis_index("s")
    w = c * NSUB + s

    @pl.when(s < 10)
    def _():
        rows = pl.ds(s * ROWS_PER_IO_SUB, ROWS_PER_IO_SUB)

        @pl.when(c == 0)
        def _():
            pltpu.sync_copy(hs_hbm.at[rows], acc_sh.at[rows])

        @pl.when(c == 1)
        def _():
            datav[0][...] = jnp.zeros((K, OUT_DIM), _f32)
            @pl.loop(0, 8)
            def _(k):
                pltpu.sync_copy(
                    datav[0].at[pl.ds(0, 125)],
                    acc_sh.at[pl.ds(s * ROWS_PER_IO_SUB + k * 125, 125)])

    _pipelined_gather_scatter(hs_hbm, src_hbm.at[w], dst_hbm.at[w], NCH2,
                              acc_sh, srcv, dstv, datav, gsems, isems, ssems)

    @pl.when(s < 10)
    def _():
        rows = pl.ds(s * ROWS_PER_IO_SUB, ROWS_PER_IO_SUB)
        pltpu.sync_copy(acc_sh.at[rows], agg_hbm.at[c].at[rows])


def _edge_aggregate2(hs, src_g2, dst_g2):
    kern = pl.kernel(
        _scatter2_body,
        out_type=jax.ShapeDtypeStruct((2, N, OUT_DIM), _f32),
        mesh=_vector_mesh(),
        scratch_types=[
            tuple(pltpu.VMEM((K,), jnp.int32) for _ in range(4)),
            tuple(pltpu.VMEM((K,), jnp.int32) for _ in range(4)),
            tuple(pltpu.VMEM((K, OUT_DIM), _f32) for _ in range(3)),
            pltpu.VMEM_SHARED((N_ACC, OUT_DIM), _f32),
            tuple(pltpu.SemaphoreType.DMA for _ in range(3)),
            tuple(pltpu.SemaphoreType.DMA for _ in range(4)),
            tuple(pltpu.SemaphoreType.DMA for _ in range(3)),
        ],
    )
    return kern(hs, src_g2, dst_g2)


def _mm1_body(x_ref, deg_ref, w_ref, out_ref):
    sc = lax.rsqrt(deg_ref[...])
    out_ref[0] = jnp.dot(x_ref[...] * sc, w_ref[...],
                         preferred_element_type=_f32,
                         precision=lax.Precision.HIGHEST)


def _mm1(x, deg2d, W1):
    grid = (N // 1000, 2)
    return pl.pallas_call(
        _mm1_body,
        grid=grid,
        in_specs=[
            pl.BlockSpec((1000, IN_DIM), lambda i, c: (i, 0)),
            pl.BlockSpec((1000, 1), lambda i, c: (i, 0)),
            pl.BlockSpec((IN_DIM, HIDDEN // 2), lambda i, c: (0, c)),
        ],
        out_specs=pl.BlockSpec((1, 1000, HIDDEN // 2), lambda i, c: (c, i, 0)),
        out_shape=jax.ShapeDtypeStruct((2, N, HIDDEN // 2), _f32),
        compiler_params=pltpu.CompilerParams(
            dimension_semantics=("parallel", "parallel")),
    )(x, deg2d, W1)


def _mm2_body(agg_ref, deg_ref, b1_ref, w_ref, out_ref):
    sc = lax.rsqrt(deg_ref[...])
    acat = jnp.concatenate([agg_ref[0], agg_ref[1]], axis=1)
    u = sc * jax.nn.relu(sc * acat + b1_ref[...])
    out_ref[...] = jnp.dot(u, w_ref[...],
                           preferred_element_type=_f32,
                           precision=lax.Precision.HIGHEST)


def _mm2(agg1, deg2d, b1, W2):
    grid = (N // 1000,)
    return pl.pallas_call(
        _mm2_body,
        grid=grid,
        in_specs=[
            pl.BlockSpec((2, 1000, HIDDEN // 2), lambda i: (0, i, 0)),
            pl.BlockSpec((1000, 1), lambda i: (i, 0)),
            pl.BlockSpec((1, HIDDEN), lambda i: (0, 0)),
            pl.BlockSpec((HIDDEN, OUT_DIM), lambda i: (0, 0)),
        ],
        out_specs=pl.BlockSpec((1000, OUT_DIM), lambda i: (i, 0)),
        out_shape=jax.ShapeDtypeStruct((N, OUT_DIM), _f32),
        compiler_params=pltpu.CompilerParams(
            dimension_semantics=("parallel",)),
    )(agg1, deg2d, b1, W2)


def _pool_body(agg_ref, deg_ref, b2_ref, brow_ref, bcol_ref,
               z_ref, zg_ref, sums_scr, cnt_scr, mx_scr):
    i = pl.program_id(0)
    nblk = pl.num_programs(0)
    sc = lax.rsqrt(deg_ref[...])
    acat = agg_ref[0] + agg_ref[1]
    z = sc * acat + b2_ref[...]
    z_ref[...] = z

    @pl.when(i == 0)
    def _():
        sums_scr[...] = jnp.zeros_like(sums_scr)
        cnt_scr[...] = jnp.zeros_like(cnt_scr)
        mx_scr[...] = jnp.full_like(mx_scr, -jnp.inf)

    brow = brow_ref[0]
    seg_ids = lax.broadcasted_iota(jnp.int32, (B, 1), 0)
    onehot = (brow == seg_ids).astype(_f32)
    sums_scr[...] += jnp.dot(onehot, z, preferred_element_type=_f32,
                             precision=lax.Precision.HIGHEST)
    cnt_scr[...] += jnp.sum(onehot, axis=1, keepdims=True)

    bcol = bcol_ref[0]
    b_lo = brow_ref[0, 0, 0]
    b_hi = brow_ref[0, 0, 999]

    def _seg_max(b, _):
        masked = jnp.where(bcol == b, z, -jnp.inf)
        row = jnp.max(masked, axis=0, keepdims=True)
        mx_scr[pl.ds(b, 1), :] = jnp.maximum(mx_scr[pl.ds(b, 1), :], row)
        return _

    lax.fori_loop(b_lo, b_hi + 1, _seg_max, None)

    @pl.when(i == nblk - 1)
    def _():
        mean = sums_scr[...] / jnp.maximum(cnt_scr[...], 1.0)
        zg_ref[:, :OUT_DIM] = mean
        zg_ref[:, OUT_DIM:] = mx_scr[...]


def _pool(agg2, deg2d, b2, brow3, bcol3):
    grid = (N // 1000,)
    return pl.pallas_call(
        _pool_body,
        grid=grid,
        in_specs=[
            pl.BlockSpec((2, 1000, OUT_DIM), lambda i: (0, i, 0)),
            pl.BlockSpec((1000, 1), lambda i: (i, 0)),
            pl.BlockSpec((1, OUT_DIM), lambda i: (0, 0)),
            pl.BlockSpec((1, 1, 1000), lambda i: (i, 0, 0)),
            pl.BlockSpec((1, 1000, 1), lambda i: (i, 0, 0)),
        ],
        out_specs=[
            pl.BlockSpec((1000, OUT_DIM), lambda i: (i, 0)),
            pl.BlockSpec((B, 2 * OUT_DIM), lambda i: (0, 0)),
        ],
        out_shape=[
            jax.ShapeDtypeStruct((N, OUT_DIM), _f32),
            jax.ShapeDtypeStruct((B, 2 * OUT_DIM), _f32),
        ],
        scratch_shapes=[
            pltpu.VMEM((B, OUT_DIM), _f32),
            pltpu.VMEM((B, 1), _f32),
            pltpu.VMEM((B, OUT_DIM), _f32),
        ],
    )(agg2, deg2d, b2, brow3, bcol3)


def kernel(x, edge_index, batch, W1, b1, W2, b2):
    pad = E_PAD - E
    pad_src = jnp.arange(pad, dtype=jnp.int32) % N
    pad_dst = SINK + (jnp.arange(pad, dtype=jnp.int32) % (N_ACC - SINK))
    src = jnp.concatenate([edge_index[0], pad_src])
    dst = jnp.concatenate([edge_index[1], pad_dst])
    src_g = src.reshape(NSUB, NCH, K)
    dst_g = dst.reshape(NSUB, NCH, K)
    src_g2 = src.reshape(2 * NSUB, NCH2, K)
    dst_g2 = dst.reshape(2 * NSUB, NCH2, K)
    dst_d = dst.reshape(NSUB, DEG_NCH, DEG_CHUNK)

    deg = _degrees(dst_d)
    deg2d = deg.reshape(N, 1)

    hs1 = _mm1(x, deg2d, W1)
    agg1 = _edge_aggregate(hs1, src_g, dst_g, HIDDEN // 2)
    hs2 = _mm2(agg1, deg2d, b1.reshape(1, HIDDEN), W2)
    agg2 = _edge_aggregate2(hs2, src_g2, dst_g2)

    brow3 = batch.reshape(N // 1000, 1, 1000)
    bcol3 = batch.reshape(N // 1000, 1000, 1)
    z, z_g = _pool(agg2, deg2d, b2.reshape(1, OUT_DIM), brow3, bcol3)
    return (z, z_g)


# speedup vs baseline: 28.7412x; 1.0308x over previous
import functools

import jax
import jax.numpy as jnp
from jax import lax
from jax.experimental import pallas as pl
from jax.experimental.pallas import tpu as pltpu
from jax.experimental.pallas import tpu_sc as plsc

N = 10000
E = 320000
IN_DIM = 128
HIDDEN = 256
OUT_DIM = 128
B = 64

NSUB = 16
K = 128
NCH = 168
E_PAD = NSUB * NCH * K
EDGES_PER_SUB = NCH * K
DEG_CHUNK = 5376
DEG_NCH = EDGES_PER_SUB // DEG_CHUNK
SINK = N
N_ACC = N + 16
ROWS_PER_IO_SUB = 1000

_f32 = jnp.float32


def _vector_mesh():
    return plsc.VectorSubcoreMesh(core_axis_name="c", subcore_axis_name="s")


def _deg_body(dst_hbm, deg_hbm, ones_v, idx_v, stage_v, acc_sh, sem):
    c = lax.axis_index("c")
    s = lax.axis_index("s")
    ones_v[...] = jnp.ones((DEG_CHUNK,), _f32)

    @pl.when((c == 0) & (s < 10))
    def _():
        pltpu.sync_copy(ones_v.at[pl.ds(0, ROWS_PER_IO_SUB)],
                        acc_sh.at[pl.ds(s * ROWS_PER_IO_SUB, ROWS_PER_IO_SUB)])

    plsc.subcore_barrier()

    @pl.when(c == 0)
    def _():
        @pl.loop(0, DEG_NCH)
        def _(j):
            pltpu.sync_copy(dst_hbm.at[s].at[j], idx_v)
            pltpu.sync_copy(ones_v, acc_sh.at[idx_v], add=True)

    plsc.subcore_barrier()

    @pl.when((c == 0) & (s < 10))
    def _():
        rows = pl.ds(s * ROWS_PER_IO_SUB, ROWS_PER_IO_SUB)
        pltpu.sync_copy(acc_sh.at[rows], stage_v)
        pltpu.sync_copy(stage_v, deg_hbm.at[rows])


def _degrees(dst_d):
    kern = pl.kernel(
        _deg_body,
        out_type=jax.ShapeDtypeStruct((N,), _f32),
        mesh=_vector_mesh(),
        scratch_types=[
            pltpu.VMEM((DEG_CHUNK,), _f32),
            pltpu.VMEM((DEG_CHUNK,), jnp.int32),
            pltpu.VMEM((ROWS_PER_IO_SUB,), _f32),
            pltpu.VMEM_SHARED((N_ACC,), _f32),
            pltpu.SemaphoreType.DMA,
        ],
    )
    return kern(dst_d)


def _pipelined_gather_scatter(table, src_rows, dst_rows, nch, acc_sh,
                              srcv, dstv, datav, gsems, isems, ssems):
    ND = 3
    NI = 4

    def idx_copies(j, bi):
        return (pltpu.make_async_copy(src_rows.at[j], srcv[bi], isems[bi]),
                pltpu.make_async_copy(dst_rows.at[j], dstv[bi], isems[bi]))

    def start_idx(j, bi):
        for cp in idx_copies(j, bi):
            cp.start()

    def wait_idx(j, bi):
        for cp in idx_copies(j, bi):
            cp.wait()

    def gather(bd, bi):
        return pltpu.make_async_copy(table.at[srcv[bi]], datav[bd], gsems[bd])

    def wait_scatter(bd):
        pltpu.make_async_copy(table.at[pl.ds(0, K)], datav[bd], ssems[bd]).wait()

    start_idx(0, 0)
    start_idx(1, 1)
    start_idx(2, 2)
    wait_idx(0, 0)
    gather(0, 0).start()
    wait_idx(1, 1)
    gather(1, 1).start()
    plsc.subcore_barrier()

    @pl.loop(0, nch, step=12)
    def _(j):
        for u in range(12):
            jj = j + u
            bd = u % ND
            bi = u % NI
            b2d = (u + 2) % ND
            b2i = (u + 2) % NI
            gather(bd, bi).wait()

            @pl.when(jj + 2 < nch)
            def _():
                wait_idx(jj + 2, b2i)

                @pl.when(jj >= 1)
                def _():
                    wait_scatter(b2d)
                gather(b2d, b2i).start()

                @pl.when(jj + 3 < nch)
                def _():
                    start_idx(jj + 3, (u + 3) % NI)

            pltpu.async_copy(datav[bd], acc_sh.at[dstv[bi]], ssems[bd],
                             add=True)

    wait_scatter(0)
    wait_scatter(1)
    wait_scatter(2)
    plsc.subcore_barrier()


def _scatter_body(half, hs_hbm, src_hbm, dst_hbm, agg_hbm,
                  srcv, dstv, datav, acc_sh, gsems, isems, ssems):
    c = lax.axis_index("c")
    s = lax.axis_index("s")
    table = hs_hbm.at[c]

    @pl.when(s < 10)
    def _():
        rows = pl.ds(s * ROWS_PER_IO_SUB, ROWS_PER_IO_SUB)
        pltpu.sync_copy(table.at[rows], acc_sh.at[rows])

    _pipelined_gather_scatter(table, src_hbm.at[s], dst_hbm.at[s], NCH, acc_sh,
                              srcv, dstv, datav, gsems, isems, ssems)

    @pl.when(s < 10)
    def _():
        rows = pl.ds(s * ROWS_PER_IO_SUB, ROWS_PER_IO_SUB)
        pltpu.sync_copy(acc_sh.at[rows], agg_hbm.at[c].at[rows])


def _edge_aggregate(hs, src_g, dst_g, half):
    kern = pl.kernel(
        functools.partial(_scatter_body, half),
        out_type=jax.ShapeDtypeStruct((2, N, half), _f32),
        mesh=_vector_mesh(),
        scratch_types=[
            tuple(pltpu.VMEM((K,), jnp.int32) for _ in range(4)),
            tuple(pltpu.VMEM((K,), jnp.int32) for _ in range(4)),
            tuple(pltpu.VMEM((K, half), _f32) for _ in range(3)),
            pltpu.VMEM_SHARED((N_ACC, half), _f32),
            tuple(pltpu.SemaphoreType.DMA for _ in range(3)),
            tuple(pltpu.SemaphoreType.DMA for _ in range(4)),
            tuple(pltpu.SemaphoreType.DMA for _ in range(3)),
        ],
    )
    return kern(hs, src_g, dst_g)


NCH2 = E_PAD // (2 * NSUB * K)


def _scatter2_body(hs_hbm, src_hbm, dst_hbm, agg_hbm,
                   srcv, dstv, datav, acc_sh, gsems, isems, ssems):
    c = lax.axis_index("c")
    s = lax.axis_index("s")
    w = c * NSUB + s

    @pl.when(s < 10)
    def _():
        rows = pl.ds(s * ROWS_PER_IO_SUB, ROWS_PER_IO_SUB)

        @pl.when(c == 0)
        def _():
            pltpu.sync_copy(hs_hbm.at[rows], acc_sh.at[rows])

        @pl.when(c == 1)
        def _():
            datav[0][...] = jnp.zeros((K, OUT_DIM), _f32)
            @pl.loop(0, 8)
            def _(k):
                pltpu.sync_copy(
                    datav[0].at[pl.ds(0, 125)],
                    acc_sh.at[pl.ds(s * ROWS_PER_IO_SUB + k * 125, 125)])

    _pipelined_gather_scatter(hs_hbm, src_hbm.at[w], dst_hbm.at[w], NCH2,
                              acc_sh, srcv, dstv, datav, gsems, isems, ssems)

    @pl.when(s < 10)
    def _():
        rows = pl.ds(s * ROWS_PER_IO_SUB, ROWS_PER_IO_SUB)
        pltpu.sync_copy(acc_sh.at[rows], agg_hbm.at[c].at[rows])


def _edge_aggregate2(hs, src_g2, dst_g2):
    kern = pl.kernel(
        _scatter2_body,
        out_type=jax.ShapeDtypeStruct((2, N, OUT_DIM), _f32),
        mesh=_vector_mesh(),
        scratch_types=[
            tuple(pltpu.VMEM((K,), jnp.int32) for _ in range(4)),
            tuple(pltpu.VMEM((K,), jnp.int32) for _ in range(4)),
            tuple(pltpu.VMEM((K, OUT_DIM), _f32) for _ in range(3)),
            pltpu.VMEM_SHARED((N_ACC, OUT_DIM), _f32),
            tuple(pltpu.SemaphoreType.DMA for _ in range(3)),
            tuple(pltpu.SemaphoreType.DMA for _ in range(4)),
            tuple(pltpu.SemaphoreType.DMA for _ in range(3)),
        ],
    )
    return kern(hs, src_g2, dst_g2)


def _mm1_body(x_ref, deg_ref, w_ref, out_ref):
    sc = lax.rsqrt(deg_ref[...])
    out_ref[0] = jnp.dot(x_ref[...] * sc, w_ref[...],
                         preferred_element_type=_f32)


def _mm1(x, deg2d, W1):
    grid = (N // 1000, 2)
    return pl.pallas_call(
        _mm1_body,
        grid=grid,
        in_specs=[
            pl.BlockSpec((1000, IN_DIM), lambda i, c: (i, 0)),
            pl.BlockSpec((1000, 1), lambda i, c: (i, 0)),
            pl.BlockSpec((IN_DIM, HIDDEN // 2), lambda i, c: (0, c)),
        ],
        out_specs=pl.BlockSpec((1, 1000, HIDDEN // 2), lambda i, c: (c, i, 0)),
        out_shape=jax.ShapeDtypeStruct((2, N, HIDDEN // 2), _f32),
        compiler_params=pltpu.CompilerParams(
            dimension_semantics=("parallel", "parallel")),
    )(x, deg2d, W1)


def _mm2_body(agg_ref, deg_ref, b1_ref, w_ref, out_ref):
    sc = lax.rsqrt(deg_ref[...])
    acat = jnp.concatenate([agg_ref[0], agg_ref[1]], axis=1)
    u = sc * jax.nn.relu(sc * acat + b1_ref[...])
    out_ref[...] = jnp.dot(u, w_ref[...],
                           preferred_element_type=_f32)


def _mm2(agg1, deg2d, b1, W2):
    grid = (N // 1000,)
    return pl.pallas_call(
        _mm2_body,
        grid=grid,
        in_specs=[
            pl.BlockSpec((2, 1000, HIDDEN // 2), lambda i: (0, i, 0)),
            pl.BlockSpec((1000, 1), lambda i: (i, 0)),
            pl.BlockSpec((1, HIDDEN), lambda i: (0, 0)),
            pl.BlockSpec((HIDDEN, OUT_DIM), lambda i: (0, 0)),
        ],
        out_specs=pl.BlockSpec((1000, OUT_DIM), lambda i: (i, 0)),
        out_shape=jax.ShapeDtypeStruct((N, OUT_DIM), _f32),
        compiler_params=pltpu.CompilerParams(
            dimension_semantics=("parallel",)),
    )(agg1, deg2d, b1, W2)


def _pool_body(agg_ref, deg_ref, b2_ref, brow_ref, bcol_ref,
               z_ref, zg_ref, sums_scr, cnt_scr, mx_scr):
    i = pl.program_id(0)
    nblk = pl.num_programs(0)
    sc = lax.rsqrt(deg_ref[...])
    acat = agg_ref[0] + agg_ref[1]
    z = sc * acat + b2_ref[...]
    z_ref[...] = z

    @pl.when(i == 0)
    def _():
        sums_scr[...] = jnp.zeros_like(sums_scr)
        cnt_scr[...] = jnp.zeros_like(cnt_scr)
        mx_scr[...] = jnp.full_like(mx_scr, -jnp.inf)

    brow = brow_ref[0]
    seg_ids = lax.broadcasted_iota(jnp.int32, (B, 1), 0)
    onehot = (brow == seg_ids).astype(_f32)
    sums_scr[...] += jnp.dot(onehot, z, preferred_element_type=_f32)
    cnt_scr[...] += jnp.sum(onehot, axis=1, keepdims=True)

    bcol = bcol_ref[0]
    b_lo = brow_ref[0, 0, 0]
    b_hi = brow_ref[0, 0, 999]

    def _seg_max(b, _):
        masked = jnp.where(bcol == b, z, -jnp.inf)
        row = jnp.max(masked, axis=0, keepdims=True)
        mx_scr[pl.ds(b, 1), :] = jnp.maximum(mx_scr[pl.ds(b, 1), :], row)
        return _

    lax.fori_loop(b_lo, b_hi + 1, _seg_max, None)

    @pl.when(i == nblk - 1)
    def _():
        mean = sums_scr[...] / jnp.maximum(cnt_scr[...], 1.0)
        zg_ref[:, :OUT_DIM] = mean
        zg_ref[:, OUT_DIM:] = mx_scr[...]


def _pool(agg2, deg2d, b2, brow3, bcol3):
    grid = (N // 1000,)
    return pl.pallas_call(
        _pool_body,
        grid=grid,
        in_specs=[
            pl.BlockSpec((2, 1000, OUT_DIM), lambda i: (0, i, 0)),
            pl.BlockSpec((1000, 1), lambda i: (i, 0)),
            pl.BlockSpec((1, OUT_DIM), lambda i: (0, 0)),
            pl.BlockSpec((1, 1, 1000), lambda i: (i, 0, 0)),
            pl.BlockSpec((1, 1000, 1), lambda i: (i, 0, 0)),
        ],
        out_specs=[
            pl.BlockSpec((1000, OUT_DIM), lambda i: (i, 0)),
            pl.BlockSpec((B, 2 * OUT_DIM), lambda i: (0, 0)),
        ],
        out_shape=[
            jax.ShapeDtypeStruct((N, OUT_DIM), _f32),
            jax.ShapeDtypeStruct((B, 2 * OUT_DIM), _f32),
        ],
        scratch_shapes=[
            pltpu.VMEM((B, OUT_DIM), _f32),
            pltpu.VMEM((B, 1), _f32),
            pltpu.VMEM((B, OUT_DIM), _f32),
        ],
    )(agg2, deg2d, b2, brow3, bcol3)


def kernel(x, edge_index, batch, W1, b1, W2, b2):
    pad = E_PAD - E
    pad_src = jnp.arange(pad, dtype=jnp.int32) % N
    pad_dst = SINK + (jnp.arange(pad, dtype=jnp.int32) % (N_ACC - SINK))
    src = jnp.concatenate([edge_index[0], pad_src])
    dst = jnp.concatenate([edge_index[1], pad_dst])
    src_g = src.reshape(NSUB, NCH, K)
    dst_g = dst.reshape(NSUB, NCH, K)
    src_g2 = src.reshape(2 * NSUB, NCH2, K)
    dst_g2 = dst.reshape(2 * NSUB, NCH2, K)
    dst_d = dst.reshape(NSUB, DEG_NCH, DEG_CHUNK)

    deg = _degrees(dst_d)
    deg2d = deg.reshape(N, 1)

    hs1 = _mm1(x, deg2d, W1)
    agg1 = _edge_aggregate(hs1, src_g, dst_g, HIDDEN // 2)
    hs2 = _mm2(agg1, deg2d, b1.reshape(1, HIDDEN), W2)
    agg2 = _edge_aggregate2(hs2, src_g2, dst_g2)

    brow3 = batch.reshape(N // 1000, 1, 1000)
    bcol3 = batch.reshape(N // 1000, 1000, 1)
    z, z_g = _pool(agg2, deg2d, b2.reshape(1, OUT_DIM), brow3, bcol3)
    return (z, z_g)
